# Initial kernel scaffold; baseline (speedup 1.0000x reference)
import jax
import jax.numpy as jnp
from jax.experimental import pallas as pl


def kernel(x, edge_index, distance_matrix, nodes_to_community, params):
    raise NotImplementedError("write your pallas kernel here")


# speedup vs baseline: 3.4676x; 3.4676x over previous
import functools

import jax
import jax.numpy as jnp
import numpy as np
from jax import lax
from jax.experimental import pallas as pl
from jax.experimental.pallas import tpu as pltpu
from jax.experimental.pallas import tpu_sc as plsc

N = 10000
E = 320000
C = 512
HEADS = 4
DHEAD = 64
SCALE = 1.0 / np.sqrt(DHEAD)

NB = 10
BR = N // NB

NCORES = 2
NSUB = 16
NW = NCORES * NSUB
EPT = E // NW
ECHUNK = 80
NCH = EPT // ECHUNK
NP = 10240
RPT = NP // NSUB

F32 = jnp.float32


def _sc_edge_agg(h_parts, src, dst):
    nparts = len(h_parts)
    z128 = jnp.zeros((ECHUNK, 128), F32)

    mesh = plsc.VectorSubcoreMesh(core_axis_name="c", subcore_axis_name="s")
    out_type = tuple(jax.ShapeDtypeStruct((NCORES, NP, 128), F32)
                     for _ in range(nparts))
    scratch = (
        pltpu.VMEM((ECHUNK,), jnp.int32),
        pltpu.VMEM((ECHUNK,), jnp.int32),
        pltpu.VMEM((ECHUNK, 128), F32),
        pltpu.VMEM((ECHUNK, 128), F32),
        pltpu.VMEM_SHARED((NP, 128), F32),
        pltpu.SemaphoreType.DMA,
    )

    @functools.partial(pl.kernel, mesh=mesh, out_type=out_type,
                       scratch_types=scratch)
    def sc_kernel(*refs):
        it = iter(refs)
        h_refs = [next(it) for _ in range(nparts)]
        src_ref = next(it)
        dst_ref = next(it)
        z128_ref = next(it)
        agg_outs = [next(it) for _ in range(nparts)]
        sidx = next(it)
        didx = next(it)
        rows = next(it)
        zbuf = next(it)
        acc = next(it)
        sem = next(it)

        c = lax.axis_index("c")
        s = lax.axis_index("s")
        wid = s * NCORES + c
        ebase = wid * EPT
        rbase = s * RPT
        nsl = RPT // ECHUNK

        pltpu.sync_copy(z128_ref, zbuf)

        for kx in range(nparts):
            for i in range(nsl):
                pltpu.sync_copy(zbuf, acc.at[pl.ds(rbase + i * ECHUNK, ECHUNK)])
            plsc.subcore_barrier()

            def body(j, carry, kx=kx):
                pltpu.sync_copy(src_ref.at[pl.ds(ebase + j * ECHUNK, ECHUNK)], sidx)
                pltpu.async_copy(h_refs[kx].at[sidx], rows, sem).wait()
                pltpu.sync_copy(dst_ref.at[pl.ds(ebase + j * ECHUNK, ECHUNK)], didx)
                pltpu.sync_copy(rows, acc.at[didx], add=True)
                return carry

            lax.fori_loop(0, NCH, body, 0)
            plsc.subcore_barrier()

            for i in range(nsl):
                r0 = rbase + i * ECHUNK
                pltpu.sync_copy(acc.at[pl.ds(r0, ECHUNK)], rows)
                pltpu.sync_copy(rows, agg_outs[kx].at[c, pl.ds(r0, ECHUNK)])
            if kx + 1 < nparts:
                plsc.subcore_barrier()

    outs = sc_kernel(*(list(h_parts) + [src, dst, z128]))
    if not isinstance(outs, (list, tuple)):
        outs = (outs,)
    return list(outs)


def _sc_deg(dst):
    z128 = jnp.zeros((ECHUNK, 128), F32)
    ones128 = jnp.ones((ECHUNK, 128), F32)

    mesh = plsc.VectorSubcoreMesh(core_axis_name="c", subcore_axis_name="s")

    @functools.partial(
        pl.kernel, mesh=mesh,
        out_type=jax.ShapeDtypeStruct((NCORES, NP, 128), F32),
        scratch_types=(
            pltpu.VMEM((ECHUNK,), jnp.int32),
            pltpu.VMEM((ECHUNK, 128), F32),
            pltpu.VMEM((ECHUNK, 128), F32),
            pltpu.VMEM_SHARED((NP, 128), F32),
        ))
    def sc_kernel(dst_ref, z128_ref, ones_ref, deg_out, didx, ones_v, zbuf,
                  degacc):
        c = lax.axis_index("c")
        s = lax.axis_index("s")
        wid = s * NCORES + c
        ebase = wid * EPT
        rbase = s * RPT
        nsl = RPT // ECHUNK

        pltpu.sync_copy(z128_ref, zbuf)
        pltpu.sync_copy(ones_ref, ones_v)
        for i in range(nsl):
            pltpu.sync_copy(zbuf, degacc.at[pl.ds(rbase + i * ECHUNK, ECHUNK)])
        plsc.subcore_barrier()

        def body(j, carry):
            pltpu.sync_copy(dst_ref.at[pl.ds(ebase + j * ECHUNK, ECHUNK)], didx)
            pltpu.sync_copy(ones_v, degacc.at[didx], add=True)
            return carry

        lax.fori_loop(0, NCH, body, 0)
        plsc.subcore_barrier()

        for i in range(nsl):
            r0 = rbase + i * ECHUNK
            pltpu.sync_copy(degacc.at[pl.ds(r0, ECHUNK)], zbuf)
            pltpu.sync_copy(zbuf, deg_out.at[c, pl.ds(r0, ECHUNK)])

    return sc_kernel(dst, z128, ones128)


def _full(shape):
    return pl.BlockSpec(shape, lambda i: tuple(0 for _ in shape))


def _dot(a, b):
    return jnp.dot(a, b, preferred_element_type=F32)


def _sage_dense(h_parts, agg_parts, deg, Wl, bl, Wr, relu):
    npart = len(h_parts)
    d_out = Wl.shape[1]
    nout = d_out // 128

    def body(*refs):
        it = iter(refs)
        h_refs = [next(it) for _ in range(npart)]
        agg_refs = [next(it) for _ in range(npart)]
        deg_ref = next(it)
        wl_ref = next(it)
        bl_ref = next(it)
        wr_ref = next(it)
        o_refs = [next(it) for _ in range(nout)]

        degs = deg_ref[0, :, 0:1] + deg_ref[1, :, 0:1]
        degc = jnp.maximum(degs, 1.0)
        acc = jnp.broadcast_to(bl_ref[...], (BR, d_out))
        for t in range(npart):
            a = agg_refs[t][...]
            mean_t = (a[0] + a[1]) / degc
            acc = acc + _dot(mean_t, wl_ref[pl.ds(t * 128, 128), :])
            acc = acc + _dot(h_refs[t][...], wr_ref[pl.ds(t * 128, 128), :])
        if relu:
            acc = jnp.maximum(acc, 0.0)
        for t in range(nout):
            o_refs[t][...] = acc[:, t * 128:(t + 1) * 128]

    d_in = 128 * npart
    in_specs = (
        [pl.BlockSpec((BR, 128), lambda i: (i, 0)) for _ in range(npart)]
        + [pl.BlockSpec((2, BR, 128), lambda i: (0, i, 0)) for _ in range(npart)]
        + [pl.BlockSpec((2, BR, 128), lambda i: (0, i, 0)),
           _full((d_in, d_out)), _full((1, d_out)), _full((d_in, d_out))]
    )
    out_specs = [pl.BlockSpec((BR, 128), lambda i: (i, 0)) for _ in range(nout)]
    out_shape = [jax.ShapeDtypeStruct((N, 128), F32) for _ in range(nout)]
    outs = pl.pallas_call(
        body, grid=(NB,), in_specs=in_specs, out_specs=out_specs,
        out_shape=out_shape,
    )(*h_parts, *agg_parts, deg, Wl, bl.reshape(1, d_out), Wr)
    return list(outs) if isinstance(outs, (list, tuple)) else [outs]


def _fcin(x, W1, b1, W2, b2):
    def body(x_ref, w1, b1r, w2, b2r, o_ref):
        h = jnp.maximum(_dot(x_ref[...], w1[...]) + b1r[...], 0.0)
        o_ref[...] = _dot(h, w2[...]) + b2r[...]

    return pl.pallas_call(
        body, grid=(NB,),
        in_specs=[pl.BlockSpec((BR, 128), lambda i: (i, 0)),
                  _full((128, 256)), _full((1, 256)),
                  _full((256, 256)), _full((1, 256))],
        out_specs=pl.BlockSpec((BR, 256), lambda i: (i, 0)),
        out_shape=jax.ShapeDtypeStruct((N, 256), F32),
    )(x, W1, b1.reshape(1, 256), W2, b2.reshape(1, 256))


def _cavg_kv(g, ids_f, Wk, bk, Wv, bv):
    def body(g_ref, ids_ref, wk, bkr, wv, bvr, k_out, v_out, logc_out,
             sums, crow, ccol):
        i = pl.program_id(0)

        @pl.when(i == 0)
        def _init():
            sums[...] = jnp.zeros_like(sums)
            crow[...] = jnp.zeros_like(crow)
            ccol[...] = jnp.zeros_like(ccol)

        ids = ids_ref[0]
        iota = lax.broadcasted_iota(jnp.int32, (BR, C), 1).astype(F32)
        oh = (ids == iota).astype(F32)
        gv = g_ref[...]
        sums[...] += lax.dot_general(oh, gv, (((0,), (0,)), ((), ())),
                                     preferred_element_type=F32)
        crow[0:1, :] += jnp.sum(oh, axis=0)[None, :]
        ccol[...] += lax.dot_general(oh, jnp.ones((BR, 8), F32),
                                     (((0,), (0,)), ((), ())),
                                     preferred_element_type=F32)

        @pl.when(i == NB - 1)
        def _fin():
            sizes = jnp.maximum(ccol[:, 0:1], 1.0)
            cavg = sums[...] / sizes
            k_out[...] = _dot(cavg, wk[...]) + bkr[...]
            v_out[...] = _dot(cavg, wv[...]) + bvr[...]
            logc_out[...] = jnp.broadcast_to(jnp.log(crow[0:1, :]), (8, C))

    return pl.pallas_call(
        body, grid=(NB,),
        in_specs=[pl.BlockSpec((BR, 256), lambda i: (i, 0)),
                  pl.BlockSpec((1, BR, 1), lambda i: (i, 0, 0)),
                  _full((256, 256)), _full((1, 256)),
                  _full((256, 256)), _full((1, 256))],
        out_specs=[_full((C, 256)), _full((C, 256)), _full((8, C))],
        out_shape=[jax.ShapeDtypeStruct((C, 256), F32),
                   jax.ShapeDtypeStruct((C, 256), F32),
                   jax.ShapeDtypeStruct((8, C), F32)],
        scratch_shapes=[pltpu.VMEM((C, 256), F32),
                        pltpu.VMEM((8, C), F32),
                        pltpu.VMEM((C, 8), F32)],
    )(g, ids_f, Wk, bk.reshape(1, 256), Wv, bv.reshape(1, 256))


def _attn_ff(g, dm, wb, kmat, vmat, logc, Wp, bp, Wq, bq, W1, b1, W2, b2):
    def body(g_ref, dm_ref, wb_ref, k_ref, v_ref, logc_ref,
             wp, bpr, wq, bqr, w1, b1r, w2, b2r, o_ref):
        gv = g_ref[...]
        qx = _dot(gv, wp[...]) + bpr[...]
        q = _dot(qx, wq[...]) + bqr[...]
        wbv = wb_ref[...]
        bias = dm_ref[...] * wbv[0:1, 0:1] + wbv[0:1, 1:2] + logc_ref[0:1, :]
        kk = k_ref[...]
        vv = v_ref[...]
        outs = []
        for h in range(HEADS):
            lo, hi = h * DHEAD, (h + 1) * DHEAD
            qh = q[:, lo:hi]
            kh = kk[:, lo:hi]
            vh = vv[:, lo:hi]
            dots = lax.dot_general(qh, kh, (((1,), (1,)), ((), ())),
                                   preferred_element_type=F32) * SCALE + bias
            m = jnp.max(dots, axis=1, keepdims=True)
            e = jnp.exp(dots - m)
            ssum = jnp.sum(e, axis=1, keepdims=True)
            outs.append(_dot(e / ssum, vh))
        o = jnp.concatenate(outs, axis=1)
        hff = jnp.maximum(_dot(o, w1[...]) + b1r[...], 0.0)
        o_ref[...] = jnp.maximum(_dot(hff, w2[...]) + b2r[...], 0.0)

    return pl.pallas_call(
        body, grid=(NB,),
        in_specs=[pl.BlockSpec((BR, 256), lambda i: (i, 0)),
                  pl.BlockSpec((BR, C), lambda i: (i, 0)),
                  _full((1, 2)),
                  _full((C, 256)), _full((C, 256)), _full((8, C)),
                  _full((256, 256)), _full((1, 256)),
                  _full((256, 256)), _full((1, 256)),
                  _full((256, 256)), _full((1, 256)),
                  _full((256, 256)), _full((1, 256))],
        out_specs=pl.BlockSpec((BR, 256), lambda i: (i, 0)),
        out_shape=jax.ShapeDtypeStruct((N, 256), F32),
    )(g, dm, wb, kmat, vmat, logc,
      Wp, bp.reshape(1, 256), Wq, bq.reshape(1, 256),
      W1, b1.reshape(1, 256), W2, b2.reshape(1, 256))


def _fcout_add(g, W, b, x_local):
    def body(g_ref, w, br, xl_ref, o_ref):
        o_ref[...] = _dot(g_ref[...], w[...]) + br[...] + xl_ref[...]

    return pl.pallas_call(
        body, grid=(NB,),
        in_specs=[pl.BlockSpec((BR, 256), lambda i: (i, 0)),
                  _full((256, 128)), _full((1, 128)),
                  pl.BlockSpec((BR, 128), lambda i: (i, 0))],
        out_specs=pl.BlockSpec((BR, 128), lambda i: (i, 0)),
        out_shape=jax.ShapeDtypeStruct((N, 128), F32),
    )(g, W, b.reshape(1, 128), x_local)


def kernel(x, edge_index, distance_matrix, nodes_to_community, params):
    src = edge_index[0]
    dst = edge_index[1]

    gnn = params['gnn']
    deg = _sc_deg(dst)
    agg0 = _sc_edge_agg([x], src, dst)
    h_parts = _sage_dense([x], agg0, deg, gnn[0]['Wl'], gnn[0]['bl'],
                          gnn[0]['Wr'], relu=True)
    agg1 = _sc_edge_agg(h_parts, src, dst)
    h_parts = _sage_dense(h_parts, agg1, deg, gnn[1]['Wl'], gnn[1]['bl'],
                          gnn[1]['Wr'], relu=True)
    agg2 = _sc_edge_agg(h_parts, src, dst)
    x_local = _sage_dense(h_parts, agg2, deg, gnn[2]['Wl'], gnn[2]['bl'],
                          gnn[2]['Wr'], relu=False)[0]

    p = params['fc_in']
    g = _fcin(x, p['W1'], p['b1'], p['W2'], p['b2'])
    ids_f = nodes_to_community.astype(F32).reshape(NB, BR, 1)
    for li in range(len(params['convs'])):
        cp = params['convs'][li]
        fp = params['ffs'][li]
        kmat, vmat, logc = _cavg_kv(g, ids_f, cp['Wk'], cp['bk'],
                                    cp['Wv'], cp['bv'])
        wb = jnp.stack([cp['w_dis'], cp['b_dis']]).reshape(1, 2)
        g = _attn_ff(g, distance_matrix, wb, kmat, vmat, logc,
                     cp['Wp'], cp['bp'], cp['Wq'], cp['bq'],
                     fp['W1'], fp['b1'], fp['W2'], fp['b2'])
    op = params['fc_out']
    return _fcout_add(g, op['W'], op['b'], x_local)


# speedup vs baseline: 7.5112x; 2.1661x over previous
import functools

import jax
import jax.numpy as jnp
import numpy as np
from jax import lax
from jax.experimental import pallas as pl
from jax.experimental.pallas import tpu as pltpu
from jax.experimental.pallas import tpu_sc as plsc

N = 10000
E = 320000
C = 512
HEADS = 4
DHEAD = 64
SCALE = 1.0 / np.sqrt(DHEAD)

NB = 10
BR = N // NB

NCORES = 2
NSUB = 16
NW = NCORES * NSUB
EPT = E // NW
ECHUNK = 80
NCH = EPT // ECHUNK
NP = 10240
RPT = NP // NSUB

F32 = jnp.float32


def _sc_edge_agg(h_parts, src, dst):
    nparts = len(h_parts)
    z128 = jnp.zeros((ECHUNK, 128), F32)

    mesh = plsc.VectorSubcoreMesh(core_axis_name="c", subcore_axis_name="s")
    out_type = tuple(jax.ShapeDtypeStruct((NCORES, NP, 128), F32)
                     for _ in range(nparts))
    scratch = (
        pltpu.VMEM((EPT,), jnp.int32),
        pltpu.VMEM((ECHUNK,), jnp.int32),
        pltpu.VMEM((ECHUNK,), jnp.int32),
        pltpu.VMEM((ECHUNK, 128), F32),
        pltpu.VMEM((ECHUNK, 128), F32),
        pltpu.VMEM((ECHUNK, 128), F32),
        pltpu.VMEM_SHARED((NP, 128), F32),
        pltpu.SemaphoreType.DMA,
        pltpu.SemaphoreType.DMA,
        pltpu.SemaphoreType.DMA,
        pltpu.SemaphoreType.DMA,
    )

    @functools.partial(pl.kernel, mesh=mesh, out_type=out_type,
                       scratch_types=scratch)
    def sc_kernel(*refs):
        it = iter(refs)
        h_refs = [next(it) for _ in range(nparts)]
        src_ref = next(it)
        dst_ref = next(it)
        z128_ref = next(it)
        agg_outs = [next(it) for _ in range(nparts)]
        sidx = next(it)
        didx_a = next(it)
        didx_b = next(it)
        rows_a = next(it)
        rows_b = next(it)
        zbuf = next(it)
        acc = next(it)
        sem_a = next(it)
        sem_b = next(it)
        sem_ia = next(it)
        sem_ib = next(it)

        c = lax.axis_index("c")
        s = lax.axis_index("s")
        wid = s * NCORES + c
        ebase = wid * EPT
        rbase = s * RPT
        nsl = RPT // ECHUNK

        pltpu.sync_copy(z128_ref, zbuf)
        pltpu.sync_copy(src_ref.at[pl.ds(ebase, EPT)], sidx)

        def dsl(j):
            return dst_ref.at[pl.ds(ebase + j * ECHUNK, ECHUNK)]

        for kx in range(nparts):
            h = h_refs[kx]
            for i in range(nsl):
                pltpu.sync_copy(zbuf, acc.at[pl.ds(rbase + i * ECHUNK, ECHUNK)])
            plsc.subcore_barrier()

            def gidx(j):
                return sidx.at[pl.ds(j * ECHUNK, ECHUNK)]

            def gather(j, buf, sem, h=h):
                pltpu.async_copy(h.at[gidx(j)], buf, sem)

            def gwait(j, buf, sem, h=h):
                pltpu.make_async_copy(h.at[gidx(j)], buf, sem).wait()

            def istage(j, dbuf, sem):
                pltpu.async_copy(dsl(j), dbuf, sem)

            def iwait(j, dbuf, sem):
                pltpu.make_async_copy(dsl(j), dbuf, sem).wait()

            def scatter(buf, dbuf):
                pltpu.sync_copy(buf, acc.at[dbuf], add=True)

            gather(0, rows_a, sem_a)
            istage(0, didx_a, sem_ia)

            def body(t, carry):
                j0 = 2 * t
                gather(j0 + 1, rows_b, sem_b)
                istage(j0 + 1, didx_b, sem_ib)
                gwait(j0, rows_a, sem_a)
                iwait(j0, didx_a, sem_ia)
                scatter(rows_a, didx_a)
                gather(j0 + 2, rows_a, sem_a)
                istage(j0 + 2, didx_a, sem_ia)
                gwait(j0 + 1, rows_b, sem_b)
                iwait(j0 + 1, didx_b, sem_ib)
                scatter(rows_b, didx_b)
                return carry

            lax.fori_loop(0, (NCH - 1) // 2, body, 0)
            gwait(NCH - 1, rows_a, sem_a)
            iwait(NCH - 1, didx_a, sem_ia)
            scatter(rows_a, didx_a)
            plsc.subcore_barrier()

            for i in range(nsl):
                r0 = rbase + i * ECHUNK
                pltpu.sync_copy(acc.at[pl.ds(r0, ECHUNK)], rows_a)
                pltpu.sync_copy(rows_a, agg_outs[kx].at[c, pl.ds(r0, ECHUNK)])
            if kx + 1 < nparts:
                plsc.subcore_barrier()

    outs = sc_kernel(*(list(h_parts) + [src, dst, z128]))
    if not isinstance(outs, (list, tuple)):
        outs = (outs,)
    return list(outs)


def _sc_deg(dst):
    z128 = jnp.zeros((ECHUNK, 128), F32)
    ones128 = jnp.ones((ECHUNK, 128), F32)

    mesh = plsc.VectorSubcoreMesh(core_axis_name="c", subcore_axis_name="s")

    @functools.partial(
        pl.kernel, mesh=mesh,
        out_type=jax.ShapeDtypeStruct((NCORES, NP, 128), F32),
        scratch_types=(
            pltpu.VMEM((ECHUNK,), jnp.int32),
            pltpu.VMEM((ECHUNK, 128), F32),
            pltpu.VMEM((ECHUNK, 128), F32),
            pltpu.VMEM_SHARED((NP, 128), F32),
        ))
    def sc_kernel(dst_ref, z128_ref, ones_ref, deg_out, dbuf, ones_v,
                  zbuf, degacc):
        c = lax.axis_index("c")
        s = lax.axis_index("s")
        wid = s * NCORES + c
        ebase = wid * EPT
        rbase = s * RPT
        nsl = RPT // ECHUNK

        pltpu.sync_copy(z128_ref, zbuf)
        pltpu.sync_copy(ones_ref, ones_v)
        for i in range(nsl):
            pltpu.sync_copy(zbuf, degacc.at[pl.ds(rbase + i * ECHUNK, ECHUNK)])
        plsc.subcore_barrier()

        def body(j, carry):
            pltpu.sync_copy(dst_ref.at[pl.ds(ebase + j * ECHUNK, ECHUNK)], dbuf)
            pltpu.sync_copy(ones_v, degacc.at[dbuf], add=True)
            return carry

        lax.fori_loop(0, NCH, body, 0)
        plsc.subcore_barrier()

        for i in range(nsl):
            r0 = rbase + i * ECHUNK
            pltpu.sync_copy(degacc.at[pl.ds(r0, ECHUNK)], zbuf)
            pltpu.sync_copy(zbuf, deg_out.at[c, pl.ds(r0, ECHUNK)])

    return sc_kernel(dst, z128, ones128)


def _full(shape):
    return pl.BlockSpec(shape, lambda i: tuple(0 for _ in shape))


def _dot(a, b):
    return jnp.dot(a, b, preferred_element_type=F32)


def _sage_dense(h_parts, agg_parts, deg, Wl, bl, Wr, relu):
    npart = len(h_parts)
    d_out = Wl.shape[1]
    nout = d_out // 128

    def body(*refs):
        it = iter(refs)
        h_refs = [next(it) for _ in range(npart)]
        agg_refs = [next(it) for _ in range(npart)]
        deg_ref = next(it)
        wl_ref = next(it)
        bl_ref = next(it)
        wr_ref = next(it)
        o_refs = [next(it) for _ in range(nout)]

        degs = deg_ref[0, :, 0:1] + deg_ref[1, :, 0:1]
        degc = jnp.maximum(degs, 1.0)
        acc = jnp.broadcast_to(bl_ref[...], (BR, d_out))
        for t in range(npart):
            a = agg_refs[t][...]
            mean_t = (a[0] + a[1]) / degc
            acc = acc + _dot(mean_t, wl_ref[pl.ds(t * 128, 128), :])
            acc = acc + _dot(h_refs[t][...], wr_ref[pl.ds(t * 128, 128), :])
        if relu:
            acc = jnp.maximum(acc, 0.0)
        for t in range(nout):
            o_refs[t][...] = acc[:, t * 128:(t + 1) * 128]

    d_in = 128 * npart
    in_specs = (
        [pl.BlockSpec((BR, 128), lambda i: (i, 0)) for _ in range(npart)]
        + [pl.BlockSpec((2, BR, 128), lambda i: (0, i, 0)) for _ in range(npart)]
        + [pl.BlockSpec((2, BR, 128), lambda i: (0, i, 0)),
           _full((d_in, d_out)), _full((1, d_out)), _full((d_in, d_out))]
    )
    out_specs = [pl.BlockSpec((BR, 128), lambda i: (i, 0)) for _ in range(nout)]
    out_shape = [jax.ShapeDtypeStruct((N, 128), F32) for _ in range(nout)]
    outs = pl.pallas_call(
        body, grid=(NB,), in_specs=in_specs, out_specs=out_specs,
        out_shape=out_shape,
    )(*h_parts, *agg_parts, deg, Wl, bl.reshape(1, d_out), Wr)
    return list(outs) if isinstance(outs, (list, tuple)) else [outs]


def _fcin(x, W1, b1, W2, b2):
    def body(x_ref, w1, b1r, w2, b2r, o_ref):
        h = jnp.maximum(_dot(x_ref[...], w1[...]) + b1r[...], 0.0)
        o_ref[...] = _dot(h, w2[...]) + b2r[...]

    return pl.pallas_call(
        body, grid=(NB,),
        in_specs=[pl.BlockSpec((BR, 128), lambda i: (i, 0)),
                  _full((128, 256)), _full((1, 256)),
                  _full((256, 256)), _full((1, 256))],
        out_specs=pl.BlockSpec((BR, 256), lambda i: (i, 0)),
        out_shape=jax.ShapeDtypeStruct((N, 256), F32),
    )(x, W1, b1.reshape(1, 256), W2, b2.reshape(1, 256))


def _cavg_kv(g, ids_f, Wk, bk, Wv, bv):
    def body(g_ref, ids_ref, wk, bkr, wv, bvr, k_out, v_out, logc_out,
             sums, crow, ccol):
        i = pl.program_id(0)

        @pl.when(i == 0)
        def _init():
            sums[...] = jnp.zeros_like(sums)
            crow[...] = jnp.zeros_like(crow)
            ccol[...] = jnp.zeros_like(ccol)

        ids = ids_ref[0]
        iota = lax.broadcasted_iota(jnp.int32, (BR, C), 1).astype(F32)
        oh = (ids == iota).astype(F32)
        gv = g_ref[...]
        sums[...] += lax.dot_general(oh, gv, (((0,), (0,)), ((), ())),
                                     preferred_element_type=F32)
        crow[0:1, :] += jnp.sum(oh, axis=0)[None, :]
        ccol[...] += lax.dot_general(oh, jnp.ones((BR, 8), F32),
                                     (((0,), (0,)), ((), ())),
                                     preferred_element_type=F32)

        @pl.when(i == NB - 1)
        def _fin():
            sizes = jnp.maximum(ccol[:, 0:1], 1.0)
            cavg = sums[...] / sizes
            k_out[...] = _dot(cavg, wk[...]) + bkr[...]
            v_out[...] = _dot(cavg, wv[...]) + bvr[...]
            logc_out[...] = jnp.broadcast_to(jnp.log(crow[0:1, :]), (8, C))

    return pl.pallas_call(
        body, grid=(NB,),
        in_specs=[pl.BlockSpec((BR, 256), lambda i: (i, 0)),
                  pl.BlockSpec((1, BR, 1), lambda i: (i, 0, 0)),
                  _full((256, 256)), _full((1, 256)),
                  _full((256, 256)), _full((1, 256))],
        out_specs=[_full((C, 256)), _full((C, 256)), _full((8, C))],
        out_shape=[jax.ShapeDtypeStruct((C, 256), F32),
                   jax.ShapeDtypeStruct((C, 256), F32),
                   jax.ShapeDtypeStruct((8, C), F32)],
        scratch_shapes=[pltpu.VMEM((C, 256), F32),
                        pltpu.VMEM((8, C), F32),
                        pltpu.VMEM((C, 8), F32)],
    )(g, ids_f, Wk, bk.reshape(1, 256), Wv, bv.reshape(1, 256))


def _attn_ff(g, dm, wb, kmat, vmat, logc, Wp, bp, Wq, bq, W1, b1, W2, b2):
    def body(g_ref, dm_ref, wb_ref, k_ref, v_ref, logc_ref,
             wp, bpr, wq, bqr, w1, b1r, w2, b2r, o_ref):
        gv = g_ref[...]
        qx = _dot(gv, wp[...]) + bpr[...]
        q = _dot(qx, wq[...]) + bqr[...]
        wbv = wb_ref[...]
        bias = dm_ref[...] * wbv[0:1, 0:1] + wbv[0:1, 1:2] + logc_ref[0:1, :]
        kk = k_ref[...]
        vv = v_ref[...]
        outs = []
        for h in range(HEADS):
            lo, hi = h * DHEAD, (h + 1) * DHEAD
            qh = q[:, lo:hi]
            kh = kk[:, lo:hi]
            vh = vv[:, lo:hi]
            dots = lax.dot_general(qh, kh, (((1,), (1,)), ((), ())),
                                   preferred_element_type=F32) * SCALE + bias
            m = jnp.max(dots, axis=1, keepdims=True)
            e = jnp.exp(dots - m)
            ssum = jnp.sum(e, axis=1, keepdims=True)
            outs.append(_dot(e / ssum, vh))
        o = jnp.concatenate(outs, axis=1)
        hff = jnp.maximum(_dot(o, w1[...]) + b1r[...], 0.0)
        o_ref[...] = jnp.maximum(_dot(hff, w2[...]) + b2r[...], 0.0)

    return pl.pallas_call(
        body, grid=(NB,),
        in_specs=[pl.BlockSpec((BR, 256), lambda i: (i, 0)),
                  pl.BlockSpec((BR, C), lambda i: (i, 0)),
                  _full((1, 2)),
                  _full((C, 256)), _full((C, 256)), _full((8, C)),
                  _full((256, 256)), _full((1, 256)),
                  _full((256, 256)), _full((1, 256)),
                  _full((256, 256)), _full((1, 256)),
                  _full((256, 256)), _full((1, 256))],
        out_specs=pl.BlockSpec((BR, 256), lambda i: (i, 0)),
        out_shape=jax.ShapeDtypeStruct((N, 256), F32),
    )(g, dm, wb, kmat, vmat, logc,
      Wp, bp.reshape(1, 256), Wq, bq.reshape(1, 256),
      W1, b1.reshape(1, 256), W2, b2.reshape(1, 256))


def _fcout_add(g, W, b, x_local):
    def body(g_ref, w, br, xl_ref, o_ref):
        o_ref[...] = _dot(g_ref[...], w[...]) + br[...] + xl_ref[...]

    return pl.pallas_call(
        body, grid=(NB,),
        in_specs=[pl.BlockSpec((BR, 256), lambda i: (i, 0)),
                  _full((256, 128)), _full((1, 128)),
                  pl.BlockSpec((BR, 128), lambda i: (i, 0))],
        out_specs=pl.BlockSpec((BR, 128), lambda i: (i, 0)),
        out_shape=jax.ShapeDtypeStruct((N, 128), F32),
    )(g, W, b.reshape(1, 128), x_local)


def kernel(x, edge_index, distance_matrix, nodes_to_community, params):
    src = edge_index[0]
    dst = edge_index[1]

    gnn = params['gnn']
    deg = _sc_deg(dst)
    agg0 = _sc_edge_agg([x], src, dst)
    h_parts = _sage_dense([x], agg0, deg, gnn[0]['Wl'], gnn[0]['bl'],
                          gnn[0]['Wr'], relu=True)
    agg1 = _sc_edge_agg(h_parts, src, dst)
    h_parts = _sage_dense(h_parts, agg1, deg, gnn[1]['Wl'], gnn[1]['bl'],
                          gnn[1]['Wr'], relu=True)
    agg2 = _sc_edge_agg(h_parts, src, dst)
    x_local = _sage_dense(h_parts, agg2, deg, gnn[2]['Wl'], gnn[2]['bl'],
                          gnn[2]['Wr'], relu=False)[0]

    p = params['fc_in']
    g = _fcin(x, p['W1'], p['b1'], p['W2'], p['b2'])
    ids_f = nodes_to_community.astype(F32).reshape(NB, BR, 1)
    for li in range(len(params['convs'])):
        cp = params['convs'][li]
        fp = params['ffs'][li]
        kmat, vmat, logc = _cavg_kv(g, ids_f, cp['Wk'], cp['bk'],
                                    cp['Wv'], cp['bv'])
        wb = jnp.stack([cp['w_dis'], cp['b_dis']]).reshape(1, 2)
        g = _attn_ff(g, distance_matrix, wb, kmat, vmat, logc,
                     cp['Wp'], cp['bp'], cp['Wq'], cp['bq'],
                     fp['W1'], fp['b1'], fp['W2'], fp['b2'])
    op = params['fc_out']
    return _fcout_add(g, op['W'], op['b'], x_local)


# speedup vs baseline: 7.9453x; 1.0578x over previous
import functools

import jax
import jax.numpy as jnp
import numpy as np
from jax import lax
from jax.experimental import pallas as pl
from jax.experimental.pallas import tpu as pltpu
from jax.experimental.pallas import tpu_sc as plsc

N = 10000
E = 320000
C = 512
HEADS = 4
DHEAD = 64
SCALE = 1.0 / np.sqrt(DHEAD)

NB = 10
BR = N // NB

NCORES = 2
NSUB = 16
NW = NCORES * NSUB
EPT = E // NW
ECHUNK = 80
NCH = EPT // ECHUNK
NP = 10240
NRING = 2
RPT = NP // NSUB

F32 = jnp.float32


def _sc_edge_agg(h_parts, src, dst):
    nparts = len(h_parts)
    z128 = jnp.zeros((ECHUNK, 128), F32)

    mesh = plsc.VectorSubcoreMesh(core_axis_name="c", subcore_axis_name="s")
    out_type = tuple(jax.ShapeDtypeStruct((NCORES, NP, 128), F32)
                     for _ in range(nparts))
    scratch = (
        pltpu.VMEM((EPT,), jnp.int32),
    ) + tuple(pltpu.VMEM((ECHUNK,), jnp.int32) for _ in range(NRING)) \
      + tuple(pltpu.VMEM((ECHUNK, 128), F32) for _ in range(NRING)) + (
        pltpu.VMEM((ECHUNK, 128), F32),
        pltpu.VMEM_SHARED((NP, 128), F32),
    ) + tuple(pltpu.SemaphoreType.DMA for _ in range(2 * NRING))

    @functools.partial(pl.kernel, mesh=mesh, out_type=out_type,
                       scratch_types=scratch)
    def sc_kernel(*refs):
        it = iter(refs)
        h_refs = [next(it) for _ in range(nparts)]
        src_ref = next(it)
        dst_ref = next(it)
        z128_ref = next(it)
        agg_outs = [next(it) for _ in range(nparts)]
        sidx = next(it)
        didxs = [next(it) for _ in range(NRING)]
        rowss = [next(it) for _ in range(NRING)]
        zbuf = next(it)
        acc = next(it)
        sems = [next(it) for _ in range(NRING)]
        isems = [next(it) for _ in range(NRING)]

        c = lax.axis_index("c")
        s = lax.axis_index("s")
        wid = s * NCORES + c
        ebase = wid * EPT
        rbase = s * RPT
        nsl = RPT // ECHUNK

        pltpu.sync_copy(z128_ref, zbuf)
        pltpu.sync_copy(src_ref.at[pl.ds(ebase, EPT)], sidx)

        def dsl(j):
            return dst_ref.at[pl.ds(ebase + j * ECHUNK, ECHUNK)]

        for kx in range(nparts):
            h = h_refs[kx]
            for i in range(nsl):
                pltpu.sync_copy(zbuf, acc.at[pl.ds(rbase + i * ECHUNK, ECHUNK)])
            plsc.subcore_barrier()

            def gidx(j):
                return sidx.at[pl.ds(j * ECHUNK, ECHUNK)]

            def issue(j, b, h=h):
                pltpu.async_copy(h.at[gidx(j)], rowss[b], sems[b])
                pltpu.async_copy(dsl(j), didxs[b], isems[b])

            def drain_scatter(j, b, h=h):
                pltpu.make_async_copy(h.at[gidx(j)], rowss[b], sems[b]).wait()
                pltpu.make_async_copy(dsl(j), didxs[b], isems[b]).wait()
                pltpu.sync_copy(rowss[b], acc.at[didxs[b]], add=True)

            for b in range(NRING):
                issue(b, b)

            def body(t, carry):
                j0 = NRING * t
                for b in range(NRING):
                    drain_scatter(j0 + b, b)
                    issue(j0 + NRING + b, b)
                return carry

            covered = NRING * (NCH // NRING)
            lax.fori_loop(0, NCH // NRING - 1, body, 0)
            for b in range(NRING):
                drain_scatter(covered - NRING + b, b)
            for j in range(covered, NCH):
                issue(j, 0)
                drain_scatter(j, 0)
            plsc.subcore_barrier()

            for i in range(nsl):
                r0 = rbase + i * ECHUNK
                pltpu.sync_copy(acc.at[pl.ds(r0, ECHUNK)], rowss[0])
                pltpu.sync_copy(rowss[0], agg_outs[kx].at[c, pl.ds(r0, ECHUNK)])
            if kx + 1 < nparts:
                plsc.subcore_barrier()

    outs = sc_kernel(*(list(h_parts) + [src, dst, z128]))
    if not isinstance(outs, (list, tuple)):
        outs = (outs,)
    return list(outs)


def _sc_deg(dst):
    z32 = jnp.zeros((ECHUNK, 128), F32)
    ones32 = jnp.ones((ECHUNK, 128), F32)

    mesh = plsc.VectorSubcoreMesh(core_axis_name="c", subcore_axis_name="s")

    @functools.partial(
        pl.kernel, mesh=mesh,
        out_type=jax.ShapeDtypeStruct((NCORES, NP, 128), F32),
        scratch_types=(
            pltpu.VMEM((ECHUNK,), jnp.int32),
            pltpu.VMEM((ECHUNK,), jnp.int32),
            pltpu.VMEM((ECHUNK, 128), F32),
            pltpu.VMEM((ECHUNK, 128), F32),
            pltpu.VMEM_SHARED((NP, 128), F32),
            pltpu.SemaphoreType.DMA,
            pltpu.SemaphoreType.DMA,
        ))
    def sc_kernel(dst_ref, z32_ref, ones_ref, deg_out, didx_a, didx_b, ones_v,
                  zbuf, degacc, sem_a, sem_b):
        c = lax.axis_index("c")
        s = lax.axis_index("s")
        wid = s * NCORES + c
        ebase = wid * EPT
        rbase = s * RPT
        nsl = RPT // ECHUNK

        def dsl(j):
            return dst_ref.at[pl.ds(ebase + j * ECHUNK, ECHUNK)]

        pltpu.sync_copy(z32_ref, zbuf)
        pltpu.sync_copy(ones_ref, ones_v)
        for i in range(nsl):
            pltpu.sync_copy(zbuf, degacc.at[pl.ds(rbase + i * ECHUNK, ECHUNK)])
        plsc.subcore_barrier()

        pltpu.async_copy(dsl(0), didx_a, sem_a)

        def body(t, carry):
            j0 = 2 * t
            pltpu.async_copy(dsl(j0 + 1), didx_b, sem_b)
            pltpu.make_async_copy(dsl(j0), didx_a, sem_a).wait()
            pltpu.sync_copy(ones_v, degacc.at[didx_a], add=True)
            pltpu.async_copy(dsl(j0 + 2), didx_a, sem_a)
            pltpu.make_async_copy(dsl(j0 + 1), didx_b, sem_b).wait()
            pltpu.sync_copy(ones_v, degacc.at[didx_b], add=True)
            return carry

        lax.fori_loop(0, (NCH - 1) // 2, body, 0)
        pltpu.make_async_copy(dsl(NCH - 1), didx_a, sem_a).wait()
        pltpu.sync_copy(ones_v, degacc.at[didx_a], add=True)
        plsc.subcore_barrier()

        for i in range(nsl):
            r0 = rbase + i * ECHUNK
            pltpu.sync_copy(degacc.at[pl.ds(r0, ECHUNK)], zbuf)
            pltpu.sync_copy(zbuf, deg_out.at[c, pl.ds(r0, ECHUNK)])

    return sc_kernel(dst, z32, ones32)


def _full(shape):
    return pl.BlockSpec(shape, lambda i: tuple(0 for _ in shape))


def _dot(a, b):
    return jnp.dot(a, b, preferred_element_type=F32)


def _sage_dense(h_parts, agg_parts, deg, Wl, bl, Wr, relu):
    npart = len(h_parts)
    d_out = Wl.shape[1]
    nout = d_out // 128

    def body(*refs):
        it = iter(refs)
        h_refs = [next(it) for _ in range(npart)]
        agg_refs = [next(it) for _ in range(npart)]
        deg_ref = next(it)
        wl_ref = next(it)
        bl_ref = next(it)
        wr_ref = next(it)
        o_refs = [next(it) for _ in range(nout)]

        degs = deg_ref[0, :, 0:1] + deg_ref[1, :, 0:1]
        degc = jnp.maximum(degs, 1.0)
        acc = jnp.broadcast_to(bl_ref[...], (BR, d_out))
        for t in range(npart):
            a = agg_refs[t][...]
            mean_t = (a[0] + a[1]) / degc
            acc = acc + _dot(mean_t, wl_ref[pl.ds(t * 128, 128), :])
            acc = acc + _dot(h_refs[t][...], wr_ref[pl.ds(t * 128, 128), :])
        if relu:
            acc = jnp.maximum(acc, 0.0)
        for t in range(nout):
            o_refs[t][...] = acc[:, t * 128:(t + 1) * 128]

    d_in = 128 * npart
    in_specs = (
        [pl.BlockSpec((BR, 128), lambda i: (i, 0)) for _ in range(npart)]
        + [pl.BlockSpec((2, BR, 128), lambda i: (0, i, 0)) for _ in range(npart)]
        + [pl.BlockSpec((2, BR, 128), lambda i: (0, i, 0)),
           _full((d_in, d_out)), _full((1, d_out)), _full((d_in, d_out))]
    )
    out_specs = [pl.BlockSpec((BR, 128), lambda i: (i, 0)) for _ in range(nout)]
    out_shape = [jax.ShapeDtypeStruct((N, 128), F32) for _ in range(nout)]
    outs = pl.pallas_call(
        body, grid=(NB,), in_specs=in_specs, out_specs=out_specs,
        out_shape=out_shape,
    )(*h_parts, *agg_parts, deg, Wl, bl.reshape(1, d_out), Wr)
    return list(outs) if isinstance(outs, (list, tuple)) else [outs]


def _fcin(x, W1, b1, W2, b2):
    def body(x_ref, w1, b1r, w2, b2r, o_ref):
        h = jnp.maximum(_dot(x_ref[...], w1[...]) + b1r[...], 0.0)
        o_ref[...] = _dot(h, w2[...]) + b2r[...]

    return pl.pallas_call(
        body, grid=(NB,),
        in_specs=[pl.BlockSpec((BR, 128), lambda i: (i, 0)),
                  _full((128, 256)), _full((1, 256)),
                  _full((256, 256)), _full((1, 256))],
        out_specs=pl.BlockSpec((BR, 256), lambda i: (i, 0)),
        out_shape=jax.ShapeDtypeStruct((N, 256), F32),
    )(x, W1, b1.reshape(1, 256), W2, b2.reshape(1, 256))


def _cavg_kv(g, ids_f, Wk, bk, Wv, bv):
    def body(g_ref, ids_ref, wk, bkr, wv, bvr, k_out, v_out, logc_out,
             sums, crow, ccol):
        i = pl.program_id(0)

        @pl.when(i == 0)
        def _init():
            sums[...] = jnp.zeros_like(sums)
            crow[...] = jnp.zeros_like(crow)
            ccol[...] = jnp.zeros_like(ccol)

        ids = ids_ref[0]
        iota = lax.broadcasted_iota(jnp.int32, (BR, C), 1).astype(F32)
        oh = (ids == iota).astype(F32)
        gv = g_ref[...]
        sums[...] += lax.dot_general(oh, gv, (((0,), (0,)), ((), ())),
                                     preferred_element_type=F32)
        crow[0:1, :] += jnp.sum(oh, axis=0)[None, :]
        ccol[...] += lax.dot_general(oh, jnp.ones((BR, 8), F32),
                                     (((0,), (0,)), ((), ())),
                                     preferred_element_type=F32)

        @pl.when(i == NB - 1)
        def _fin():
            sizes = jnp.maximum(ccol[:, 0:1], 1.0)
            cavg = sums[...] / sizes
            k_out[...] = _dot(cavg, wk[...]) + bkr[...]
            v_out[...] = _dot(cavg, wv[...]) + bvr[...]
            logc_out[...] = jnp.broadcast_to(jnp.log(crow[0:1, :]), (8, C))

    return pl.pallas_call(
        body, grid=(NB,),
        in_specs=[pl.BlockSpec((BR, 256), lambda i: (i, 0)),
                  pl.BlockSpec((1, BR, 1), lambda i: (i, 0, 0)),
                  _full((256, 256)), _full((1, 256)),
                  _full((256, 256)), _full((1, 256))],
        out_specs=[_full((C, 256)), _full((C, 256)), _full((8, C))],
        out_shape=[jax.ShapeDtypeStruct((C, 256), F32),
                   jax.ShapeDtypeStruct((C, 256), F32),
                   jax.ShapeDtypeStruct((8, C), F32)],
        scratch_shapes=[pltpu.VMEM((C, 256), F32),
                        pltpu.VMEM((8, C), F32),
                        pltpu.VMEM((C, 8), F32)],
    )(g, ids_f, Wk, bk.reshape(1, 256), Wv, bv.reshape(1, 256))


def _attn_ff(g, dm, wb, kmat, vmat, logc, Wp, bp, Wq, bq, W1, b1, W2, b2):
    def body(g_ref, dm_ref, wb_ref, k_ref, v_ref, logc_ref,
             wp, bpr, wq, bqr, w1, b1r, w2, b2r, o_ref):
        gv = g_ref[...]
        qx = _dot(gv, wp[...]) + bpr[...]
        q = _dot(qx, wq[...]) + bqr[...]
        wbv = wb_ref[...]
        bias = dm_ref[...] * wbv[0:1, 0:1] + wbv[0:1, 1:2] + logc_ref[0:1, :]
        kk = k_ref[...]
        vv = v_ref[...]
        outs = []
        for h in range(HEADS):
            lo, hi = h * DHEAD, (h + 1) * DHEAD
            qh = q[:, lo:hi]
            kh = kk[:, lo:hi]
            vh = vv[:, lo:hi]
            dots = lax.dot_general(qh, kh, (((1,), (1,)), ((), ())),
                                   preferred_element_type=F32) * SCALE + bias
            m = jnp.max(dots, axis=1, keepdims=True)
            e = jnp.exp(dots - m)
            ssum = jnp.sum(e, axis=1, keepdims=True)
            outs.append(_dot(e / ssum, vh))
        o = jnp.concatenate(outs, axis=1)
        hff = jnp.maximum(_dot(o, w1[...]) + b1r[...], 0.0)
        o_ref[...] = jnp.maximum(_dot(hff, w2[...]) + b2r[...], 0.0)

    return pl.pallas_call(
        body, grid=(NB,),
        in_specs=[pl.BlockSpec((BR, 256), lambda i: (i, 0)),
                  pl.BlockSpec((BR, C), lambda i: (i, 0)),
                  _full((1, 2)),
                  _full((C, 256)), _full((C, 256)), _full((8, C)),
                  _full((256, 256)), _full((1, 256)),
                  _full((256, 256)), _full((1, 256)),
                  _full((256, 256)), _full((1, 256)),
                  _full((256, 256)), _full((1, 256))],
        out_specs=pl.BlockSpec((BR, 256), lambda i: (i, 0)),
        out_shape=jax.ShapeDtypeStruct((N, 256), F32),
    )(g, dm, wb, kmat, vmat, logc,
      Wp, bp.reshape(1, 256), Wq, bq.reshape(1, 256),
      W1, b1.reshape(1, 256), W2, b2.reshape(1, 256))


def _fcout_add(g, W, b, x_local):
    def body(g_ref, w, br, xl_ref, o_ref):
        o_ref[...] = _dot(g_ref[...], w[...]) + br[...] + xl_ref[...]

    return pl.pallas_call(
        body, grid=(NB,),
        in_specs=[pl.BlockSpec((BR, 256), lambda i: (i, 0)),
                  _full((256, 128)), _full((1, 128)),
                  pl.BlockSpec((BR, 128), lambda i: (i, 0))],
        out_specs=pl.BlockSpec((BR, 128), lambda i: (i, 0)),
        out_shape=jax.ShapeDtypeStruct((N, 128), F32),
    )(g, W, b.reshape(1, 128), x_local)


def kernel(x, edge_index, distance_matrix, nodes_to_community, params):
    src = edge_index[0]
    dst = edge_index[1]

    gnn = params['gnn']
    deg = _sc_deg(dst)
    agg0 = _sc_edge_agg([x], src, dst)
    h_parts = _sage_dense([x], agg0, deg, gnn[0]['Wl'], gnn[0]['bl'],
                          gnn[0]['Wr'], relu=True)
    agg1 = _sc_edge_agg(h_parts, src, dst)
    h_parts = _sage_dense(h_parts, agg1, deg, gnn[1]['Wl'], gnn[1]['bl'],
                          gnn[1]['Wr'], relu=True)
    agg2 = _sc_edge_agg(h_parts, src, dst)
    x_local = _sage_dense(h_parts, agg2, deg, gnn[2]['Wl'], gnn[2]['bl'],
                          gnn[2]['Wr'], relu=False)[0]

    p = params['fc_in']
    g = _fcin(x, p['W1'], p['b1'], p['W2'], p['b2'])
    ids_f = nodes_to_community.astype(F32).reshape(NB, BR, 1)
    for li in range(len(params['convs'])):
        cp = params['convs'][li]
        fp = params['ffs'][li]
        kmat, vmat, logc = _cavg_kv(g, ids_f, cp['Wk'], cp['bk'],
                                    cp['Wv'], cp['bv'])
        wb = jnp.stack([cp['w_dis'], cp['b_dis']]).reshape(1, 2)
        g = _attn_ff(g, distance_matrix, wb, kmat, vmat, logc,
                     cp['Wp'], cp['bp'], cp['Wq'], cp['bq'],
                     fp['W1'], fp['b1'], fp['W2'], fp['b2'])
    op = params['fc_out']
    return _fcout_add(g, op['W'], op['b'], x_local)


# speedup vs baseline: 8.4895x; 1.0685x over previous
import functools

import jax
import jax.numpy as jnp
import numpy as np
from jax import lax
from jax.experimental import pallas as pl
from jax.experimental.pallas import tpu as pltpu
from jax.experimental.pallas import tpu_sc as plsc

N = 10000
E = 320000
C = 512
HEADS = 4
DHEAD = 64
SCALE = 1.0 / np.sqrt(DHEAD)

NB = 10
BR = N // NB

NCORES = 2
NSUB = 16
NW = NCORES * NSUB
EPT = E // NW
ECHUNK = 80
NCH = EPT // ECHUNK
NP = 10240
NRING = 2
RPT = NP // NSUB

F32 = jnp.float32


def _sc_edge_agg(h_parts, src, dst, width=128):
    nparts = len(h_parts)
    zrow = jnp.zeros((ECHUNK, width), F32)

    mesh = plsc.VectorSubcoreMesh(core_axis_name="c", subcore_axis_name="s")
    out_type = tuple(jax.ShapeDtypeStruct((NCORES, NP, width), F32)
                     for _ in range(nparts))
    scratch = (
        pltpu.VMEM((EPT,), jnp.int32),
    ) + tuple(pltpu.VMEM((ECHUNK,), jnp.int32) for _ in range(NRING)) \
      + tuple(pltpu.VMEM((ECHUNK, width), F32) for _ in range(NRING)) + (
        pltpu.VMEM((ECHUNK, width), F32),
        pltpu.VMEM_SHARED((NP, width), F32),
    ) + tuple(pltpu.SemaphoreType.DMA for _ in range(2 * NRING))

    @functools.partial(pl.kernel, mesh=mesh, out_type=out_type,
                       scratch_types=scratch)
    def sc_kernel(*refs):
        it = iter(refs)
        h_refs = [next(it) for _ in range(nparts)]
        src_ref = next(it)
        dst_ref = next(it)
        z_ref = next(it)
        agg_outs = [next(it) for _ in range(nparts)]
        sidx = next(it)
        didxs = [next(it) for _ in range(NRING)]
        rowss = [next(it) for _ in range(NRING)]
        zbuf = next(it)
        acc = next(it)
        sems = [next(it) for _ in range(NRING)]
        isems = [next(it) for _ in range(NRING)]

        c = lax.axis_index("c")
        s = lax.axis_index("s")
        wid = s * NCORES + c
        ebase = wid * EPT
        rbase = s * RPT
        nsl = RPT // ECHUNK

        pltpu.sync_copy(z_ref, zbuf)
        pltpu.sync_copy(src_ref.at[pl.ds(ebase, EPT)], sidx)

        def dsl(j):
            return dst_ref.at[pl.ds(ebase + j * ECHUNK, ECHUNK)]

        for kx in range(nparts):
            h = h_refs[kx]
            for i in range(nsl):
                pltpu.sync_copy(zbuf, acc.at[pl.ds(rbase + i * ECHUNK, ECHUNK)])
            plsc.subcore_barrier()

            def gidx(j):
                return sidx.at[pl.ds(j * ECHUNK, ECHUNK)]

            def issue(j, b, h=h):
                pltpu.async_copy(h.at[gidx(j)], rowss[b], sems[b])
                pltpu.async_copy(dsl(j), didxs[b], isems[b])

            def drain_scatter(j, b, h=h):
                pltpu.make_async_copy(h.at[gidx(j)], rowss[b], sems[b]).wait()
                pltpu.make_async_copy(dsl(j), didxs[b], isems[b]).wait()
                pltpu.sync_copy(rowss[b], acc.at[didxs[b]], add=True)

            for b in range(NRING):
                issue(b, b)

            def body(t, carry):
                j0 = NRING * t
                for b in range(NRING):
                    drain_scatter(j0 + b, b)
                    issue(j0 + NRING + b, b)
                return carry

            covered = NRING * (NCH // NRING)
            lax.fori_loop(0, NCH // NRING - 1, body, 0)
            for b in range(NRING):
                drain_scatter(covered - NRING + b, b)
            for j in range(covered, NCH):
                issue(j, 0)
                drain_scatter(j, 0)
            plsc.subcore_barrier()

            for i in range(nsl):
                r0 = rbase + i * ECHUNK
                pltpu.sync_copy(acc.at[pl.ds(r0, ECHUNK)], rowss[0])
                pltpu.sync_copy(rowss[0], agg_outs[kx].at[c, pl.ds(r0, ECHUNK)])
            if kx + 1 < nparts:
                plsc.subcore_barrier()

    outs = sc_kernel(*(list(h_parts) + [src, dst, zrow]))
    if not isinstance(outs, (list, tuple)):
        outs = (outs,)
    return list(outs)


def _sc_deg(dst):
    z128 = jnp.zeros((ECHUNK, 128), F32)
    ones128 = jnp.ones((ECHUNK, 128), F32)

    mesh = plsc.VectorSubcoreMesh(core_axis_name="c", subcore_axis_name="s")

    @functools.partial(
        pl.kernel, mesh=mesh,
        out_type=jax.ShapeDtypeStruct((NCORES, NP, 128), F32),
        scratch_types=(
            pltpu.VMEM((ECHUNK,), jnp.int32),
            pltpu.VMEM((ECHUNK,), jnp.int32),
            pltpu.VMEM((ECHUNK, 128), F32),
            pltpu.VMEM((ECHUNK, 128), F32),
            pltpu.VMEM_SHARED((NP, 128), F32),
            pltpu.SemaphoreType.DMA,
            pltpu.SemaphoreType.DMA,
        ))
    def sc_kernel(dst_ref, z128_ref, ones_ref, deg_out, didx_a, didx_b, ones_v,
                  zbuf, degacc, sem_a, sem_b):
        c = lax.axis_index("c")
        s = lax.axis_index("s")
        wid = s * NCORES + c
        ebase = wid * EPT
        rbase = s * RPT
        nsl = RPT // ECHUNK

        def dsl(j):
            return dst_ref.at[pl.ds(ebase + j * ECHUNK, ECHUNK)]

        pltpu.sync_copy(z128_ref, zbuf)
        pltpu.sync_copy(ones_ref, ones_v)
        for i in range(nsl):
            pltpu.sync_copy(zbuf, degacc.at[pl.ds(rbase + i * ECHUNK, ECHUNK)])
        plsc.subcore_barrier()

        pltpu.async_copy(dsl(0), didx_a, sem_a)

        def body(t, carry):
            j0 = 2 * t
            pltpu.async_copy(dsl(j0 + 1), didx_b, sem_b)
            pltpu.make_async_copy(dsl(j0), didx_a, sem_a).wait()
            pltpu.sync_copy(ones_v, degacc.at[didx_a], add=True)
            pltpu.async_copy(dsl(j0 + 2), didx_a, sem_a)
            pltpu.make_async_copy(dsl(j0 + 1), didx_b, sem_b).wait()
            pltpu.sync_copy(ones_v, degacc.at[didx_b], add=True)
            return carry

        lax.fori_loop(0, (NCH - 1) // 2, body, 0)
        pltpu.make_async_copy(dsl(NCH - 1), didx_a, sem_a).wait()
        pltpu.sync_copy(ones_v, degacc.at[didx_a], add=True)
        plsc.subcore_barrier()

        for i in range(nsl):
            r0 = rbase + i * ECHUNK
            pltpu.sync_copy(degacc.at[pl.ds(r0, ECHUNK)], zbuf)
            pltpu.sync_copy(zbuf, deg_out.at[c, pl.ds(r0, ECHUNK)])

    return sc_kernel(dst, z128, ones128)


def _full(shape):
    return pl.BlockSpec(shape, lambda i: tuple(0 for _ in shape))


def _dot(a, b):
    return jnp.dot(a, b, preferred_element_type=F32)


def _sage_dense(h_parts, agg_parts, deg_src, Wl, bl, Wr, relu,
                pre_projected=False):
    npart = len(h_parts)
    nagg = len(agg_parts)
    aw = [a.shape[2] for a in agg_parts]
    d_out = Wl.shape[1] if not pre_projected else 128
    nout = d_out // 128

    def body(*refs):
        it = iter(refs)
        h_refs = [next(it) for _ in range(npart)]
        agg_refs = [next(it) for _ in range(nagg)]
        deg_ref = next(it)
        wl_ref = next(it) if not pre_projected else None
        bl_ref = next(it)
        wr_ref = next(it)
        o_refs = [next(it) for _ in range(nout)]

        degs = deg_ref[0, :, 0:1] + deg_ref[1, :, 0:1]
        degc = jnp.maximum(degs, 1.0)
        acc = jnp.broadcast_to(bl_ref[...], (BR, d_out))
        for t in range(nagg):
            a = agg_refs[t][...]
            mean_t = (a[0, :, :128] + a[1, :, :128]) / degc
            if pre_projected:
                acc = acc + mean_t
            else:
                acc = acc + _dot(mean_t, wl_ref[pl.ds(t * 128, 128), :])
        for t in range(npart):
            acc = acc + _dot(h_refs[t][...], wr_ref[pl.ds(t * 128, 128), :])
        if relu:
            acc = jnp.maximum(acc, 0.0)
        for t in range(nout):
            o_refs[t][...] = acc[:, t * 128:(t + 1) * 128]

    d_in = 128 * npart
    in_specs = (
        [pl.BlockSpec((BR, 128), lambda i: (i, 0)) for _ in range(npart)]
        + [pl.BlockSpec((2, BR, w), lambda i: (0, i, 0)) for w in aw]
        + [pl.BlockSpec((2, BR, 128), lambda i: (0, i, 0))]
        + ([_full((d_in, d_out))] if not pre_projected else [])
        + [_full((1, d_out)), _full((d_in, d_out))]
    )
    out_specs = [pl.BlockSpec((BR, 128), lambda i: (i, 0)) for _ in range(nout)]
    out_shape = [jax.ShapeDtypeStruct((N, 128), F32) for _ in range(nout)]
    args = (list(h_parts) + list(agg_parts) + [deg_src]
            + ([Wl] if not pre_projected else [])
            + [bl.reshape(1, d_out), Wr])
    outs = pl.pallas_call(
        body, grid=(NB,), in_specs=in_specs, out_specs=out_specs,
        out_shape=out_shape,
    )(*args)
    return list(outs) if isinstance(outs, (list, tuple)) else [outs]


def _matmul128(h_parts, W):
    npart = len(h_parts)

    def body(*refs):
        it = iter(refs)
        h_refs = [next(it) for _ in range(npart)]
        w_ref = next(it)
        o_ref = next(it)
        acc = jnp.zeros((BR, 128), F32)
        for t in range(npart):
            acc = acc + _dot(h_refs[t][...], w_ref[pl.ds(t * 128, 128), :])
        o_ref[...] = acc

    return pl.pallas_call(
        body, grid=(NB,),
        in_specs=[pl.BlockSpec((BR, 128), lambda i: (i, 0))
                  for _ in range(npart)] + [_full((128 * npart, 128))],
        out_specs=pl.BlockSpec((BR, 128), lambda i: (i, 0)),
        out_shape=jax.ShapeDtypeStruct((N, 128), F32),
    )(*h_parts, W)


def _fcin(x, W1, b1, W2, b2):
    def body(x_ref, w1, b1r, w2, b2r, o_ref):
        h = jnp.maximum(_dot(x_ref[...], w1[...]) + b1r[...], 0.0)
        o_ref[...] = _dot(h, w2[...]) + b2r[...]

    return pl.pallas_call(
        body, grid=(NB,),
        in_specs=[pl.BlockSpec((BR, 128), lambda i: (i, 0)),
                  _full((128, 256)), _full((1, 256)),
                  _full((256, 256)), _full((1, 256))],
        out_specs=pl.BlockSpec((BR, 256), lambda i: (i, 0)),
        out_shape=jax.ShapeDtypeStruct((N, 256), F32),
    )(x, W1, b1.reshape(1, 256), W2, b2.reshape(1, 256))


def _cavg_kv(g, ids_f, Wk, bk, Wv, bv):
    def body(g_ref, ids_ref, wk, bkr, wv, bvr, k_out, v_out, logc_out,
             sums, crow, ccol):
        i = pl.program_id(0)

        @pl.when(i == 0)
        def _init():
            sums[...] = jnp.zeros_like(sums)
            crow[...] = jnp.zeros_like(crow)
            ccol[...] = jnp.zeros_like(ccol)

        ids = ids_ref[0]
        iota = lax.broadcasted_iota(jnp.int32, (BR, C), 1).astype(F32)
        oh = (ids == iota).astype(F32)
        gv = g_ref[...]
        sums[...] += lax.dot_general(oh, gv, (((0,), (0,)), ((), ())),
                                     preferred_element_type=F32)
        crow[0:1, :] += jnp.sum(oh, axis=0)[None, :]
        ccol[...] += lax.dot_general(oh, jnp.ones((BR, 8), F32),
                                     (((0,), (0,)), ((), ())),
                                     preferred_element_type=F32)

        @pl.when(i == NB - 1)
        def _fin():
            sizes = jnp.maximum(ccol[:, 0:1], 1.0)
            cavg = sums[...] / sizes
            k_out[...] = _dot(cavg, wk[...]) + bkr[...]
            v_out[...] = _dot(cavg, wv[...]) + bvr[...]
            logc_out[...] = jnp.broadcast_to(jnp.log(crow[0:1, :]), (8, C))

    return pl.pallas_call(
        body, grid=(NB,),
        in_specs=[pl.BlockSpec((BR, 256), lambda i: (i, 0)),
                  pl.BlockSpec((1, BR, 1), lambda i: (i, 0, 0)),
                  _full((256, 256)), _full((1, 256)),
                  _full((256, 256)), _full((1, 256))],
        out_specs=[_full((C, 256)), _full((C, 256)), _full((8, C))],
        out_shape=[jax.ShapeDtypeStruct((C, 256), F32),
                   jax.ShapeDtypeStruct((C, 256), F32),
                   jax.ShapeDtypeStruct((8, C), F32)],
        scratch_shapes=[pltpu.VMEM((C, 256), F32),
                        pltpu.VMEM((8, C), F32),
                        pltpu.VMEM((C, 8), F32)],
    )(g, ids_f, Wk, bk.reshape(1, 256), Wv, bv.reshape(1, 256))


def _attn_ff(g, dm, wb, kmat, vmat, logc, Wp, bp, Wq, bq, W1, b1, W2, b2):
    def body(g_ref, dm_ref, wb_ref, k_ref, v_ref, logc_ref,
             wp, bpr, wq, bqr, w1, b1r, w2, b2r, o_ref):
        gv = g_ref[...]
        qx = _dot(gv, wp[...]) + bpr[...]
        q = _dot(qx, wq[...]) + bqr[...]
        wbv = wb_ref[...]
        bias = dm_ref[...] * wbv[0:1, 0:1] + wbv[0:1, 1:2] + logc_ref[0:1, :]
        kk = k_ref[...]
        vv = v_ref[...]
        outs = []
        for h in range(HEADS):
            lo, hi = h * DHEAD, (h + 1) * DHEAD
            qh = q[:, lo:hi]
            kh = kk[:, lo:hi]
            vh = vv[:, lo:hi]
            dots = lax.dot_general(qh, kh, (((1,), (1,)), ((), ())),
                                   preferred_element_type=F32) * SCALE + bias
            m = jnp.max(dots, axis=1, keepdims=True)
            e = jnp.exp(dots - m)
            ssum = jnp.sum(e, axis=1, keepdims=True)
            outs.append(_dot(e / ssum, vh))
        o = jnp.concatenate(outs, axis=1)
        hff = jnp.maximum(_dot(o, w1[...]) + b1r[...], 0.0)
        o_ref[...] = jnp.maximum(_dot(hff, w2[...]) + b2r[...], 0.0)

    return pl.pallas_call(
        body, grid=(NB,),
        in_specs=[pl.BlockSpec((BR, 256), lambda i: (i, 0)),
                  pl.BlockSpec((BR, C), lambda i: (i, 0)),
                  _full((1, 2)),
                  _full((C, 256)), _full((C, 256)), _full((8, C)),
                  _full((256, 256)), _full((1, 256)),
                  _full((256, 256)), _full((1, 256)),
                  _full((256, 256)), _full((1, 256)),
                  _full((256, 256)), _full((1, 256))],
        out_specs=pl.BlockSpec((BR, 256), lambda i: (i, 0)),
        out_shape=jax.ShapeDtypeStruct((N, 256), F32),
    )(g, dm, wb, kmat, vmat, logc,
      Wp, bp.reshape(1, 256), Wq, bq.reshape(1, 256),
      W1, b1.reshape(1, 256), W2, b2.reshape(1, 256))


def _fcout_add(g, W, b, x_local):
    def body(g_ref, w, br, xl_ref, o_ref):
        o_ref[...] = _dot(g_ref[...], w[...]) + br[...] + xl_ref[...]

    return pl.pallas_call(
        body, grid=(NB,),
        in_specs=[pl.BlockSpec((BR, 256), lambda i: (i, 0)),
                  _full((256, 128)), _full((1, 128)),
                  pl.BlockSpec((BR, 128), lambda i: (i, 0))],
        out_specs=pl.BlockSpec((BR, 128), lambda i: (i, 0)),
        out_shape=jax.ShapeDtypeStruct((N, 128), F32),
    )(g, W, b.reshape(1, 128), x_local)


def kernel(x, edge_index, distance_matrix, nodes_to_community, params):
    src = edge_index[0]
    dst = edge_index[1]

    gnn = params['gnn']
    deg = _sc_deg(dst)
    agg0 = _sc_edge_agg([x], src, dst)
    h_parts = _sage_dense([x], agg0, deg, gnn[0]['Wl'], gnn[0]['bl'],
                          gnn[0]['Wr'], relu=True)
    agg1 = _sc_edge_agg(h_parts, src, dst)
    h_parts = _sage_dense(h_parts, agg1, deg, gnn[1]['Wl'], gnn[1]['bl'],
                          gnn[1]['Wr'], relu=True)
    p2 = _matmul128(h_parts, gnn[2]['Wl'])
    agg2 = _sc_edge_agg([p2], src, dst)
    x_local = _sage_dense(h_parts, agg2, deg, None, gnn[2]['bl'],
                          gnn[2]['Wr'], relu=False, pre_projected=True)[0]

    p = params['fc_in']
    g = _fcin(x, p['W1'], p['b1'], p['W2'], p['b2'])
    ids_f = nodes_to_community.astype(F32).reshape(NB, BR, 1)
    for li in range(len(params['convs'])):
        cp = params['convs'][li]
        fp = params['ffs'][li]
        kmat, vmat, logc = _cavg_kv(g, ids_f, cp['Wk'], cp['bk'],
                                    cp['Wv'], cp['bv'])
        wb = jnp.stack([cp['w_dis'], cp['b_dis']]).reshape(1, 2)
        g = _attn_ff(g, distance_matrix, wb, kmat, vmat, logc,
                     cp['Wp'], cp['bp'], cp['Wq'], cp['bq'],
                     fp['W1'], fp['b1'], fp['W2'], fp['b2'])
    op = params['fc_out']
    return _fcout_add(g, op['W'], op['b'], x_local)


# speedup vs baseline: 8.5878x; 1.0116x over previous
import functools

import jax
import jax.numpy as jnp
import numpy as np
from jax import lax
from jax.experimental import pallas as pl
from jax.experimental.pallas import tpu as pltpu
from jax.experimental.pallas import tpu_sc as plsc

N = 10000
E = 320000
C = 512
HEADS = 4
DHEAD = 64
SCALE = 1.0 / np.sqrt(DHEAD)

NB = 10
BR = N // NB

NCORES = 2
NSUB = 16
NW = NCORES * NSUB
EPT = E // NW
ECHUNK = 80
NCH = EPT // ECHUNK
NP = 10240
NRING = 2
RPT = NP // NSUB

F32 = jnp.float32


def _sc_edge_agg(h_parts, src, dst, width=128):
    nparts = len(h_parts)
    zrow = jnp.zeros((ECHUNK, width), F32)

    mesh = plsc.VectorSubcoreMesh(core_axis_name="c", subcore_axis_name="s")
    out_type = tuple(jax.ShapeDtypeStruct((NCORES, NP, width), F32)
                     for _ in range(nparts))
    scratch = (
        pltpu.VMEM((EPT,), jnp.int32),
    ) + tuple(pltpu.VMEM((ECHUNK,), jnp.int32) for _ in range(NRING)) \
      + tuple(pltpu.VMEM((ECHUNK, width), F32) for _ in range(NRING)) + (
        pltpu.VMEM((ECHUNK, width), F32),
        pltpu.VMEM_SHARED((NP, width), F32),
    ) + tuple(pltpu.SemaphoreType.DMA for _ in range(2 * NRING))

    @functools.partial(pl.kernel, mesh=mesh, out_type=out_type,
                       scratch_types=scratch)
    def sc_kernel(*refs):
        it = iter(refs)
        h_refs = [next(it) for _ in range(nparts)]
        src_ref = next(it)
        dst_ref = next(it)
        z_ref = next(it)
        agg_outs = [next(it) for _ in range(nparts)]
        sidx = next(it)
        didxs = [next(it) for _ in range(NRING)]
        rowss = [next(it) for _ in range(NRING)]
        zbuf = next(it)
        acc = next(it)
        sems = [next(it) for _ in range(NRING)]
        isems = [next(it) for _ in range(NRING)]

        c = lax.axis_index("c")
        s = lax.axis_index("s")
        wid = s * NCORES + c
        ebase = wid * EPT
        rbase = s * RPT
        nsl = RPT // ECHUNK

        pltpu.sync_copy(z_ref, zbuf)
        pltpu.sync_copy(src_ref.at[pl.ds(ebase, EPT)], sidx)

        def dsl(j):
            return dst_ref.at[pl.ds(ebase + j * ECHUNK, ECHUNK)]

        for kx in range(nparts):
            h = h_refs[kx]
            for i in range(nsl):
                pltpu.sync_copy(zbuf, acc.at[pl.ds(rbase + i * ECHUNK, ECHUNK)])
            plsc.subcore_barrier()

            def gidx(j):
                return sidx.at[pl.ds(j * ECHUNK, ECHUNK)]

            def issue(j, b, h=h):
                pltpu.async_copy(h.at[gidx(j)], rowss[b], sems[b])
                pltpu.async_copy(dsl(j), didxs[b], isems[b])

            def drain_scatter(j, b, h=h):
                pltpu.make_async_copy(h.at[gidx(j)], rowss[b], sems[b]).wait()
                pltpu.make_async_copy(dsl(j), didxs[b], isems[b]).wait()
                pltpu.sync_copy(rowss[b], acc.at[didxs[b]], add=True)

            for b in range(NRING):
                issue(b, b)

            def body(t, carry):
                j0 = NRING * t
                for b in range(NRING):
                    drain_scatter(j0 + b, b)
                    issue(j0 + NRING + b, b)
                return carry

            covered = NRING * (NCH // NRING)
            lax.fori_loop(0, NCH // NRING - 1, body, 0)
            for b in range(NRING):
                drain_scatter(covered - NRING + b, b)
            for j in range(covered, NCH):
                issue(j, 0)
                drain_scatter(j, 0)
            plsc.subcore_barrier()

            for i in range(nsl):
                r0 = rbase + i * ECHUNK
                pltpu.sync_copy(acc.at[pl.ds(r0, ECHUNK)], rowss[0])
                pltpu.sync_copy(rowss[0], agg_outs[kx].at[c, pl.ds(r0, ECHUNK)])
            if kx + 1 < nparts:
                plsc.subcore_barrier()

    outs = sc_kernel(*(list(h_parts) + [src, dst, zrow]))
    if not isinstance(outs, (list, tuple)):
        outs = (outs,)
    return list(outs)


def _sc_deg(dst):
    z128 = jnp.zeros((ECHUNK, 128), F32)
    ones128 = jnp.ones((ECHUNK, 128), F32)

    mesh = plsc.VectorSubcoreMesh(core_axis_name="c", subcore_axis_name="s")

    @functools.partial(
        pl.kernel, mesh=mesh,
        out_type=jax.ShapeDtypeStruct((NCORES, NP, 128), F32),
        scratch_types=(
            pltpu.VMEM((ECHUNK,), jnp.int32),
            pltpu.VMEM((ECHUNK,), jnp.int32),
            pltpu.VMEM((ECHUNK, 128), F32),
            pltpu.VMEM((ECHUNK, 128), F32),
            pltpu.VMEM_SHARED((NP, 128), F32),
            pltpu.SemaphoreType.DMA,
            pltpu.SemaphoreType.DMA,
        ))
    def sc_kernel(dst_ref, z128_ref, ones_ref, deg_out, didx_a, didx_b, ones_v,
                  zbuf, degacc, sem_a, sem_b):
        c = lax.axis_index("c")
        s = lax.axis_index("s")
        wid = s * NCORES + c
        ebase = wid * EPT
        rbase = s * RPT
        nsl = RPT // ECHUNK

        def dsl(j):
            return dst_ref.at[pl.ds(ebase + j * ECHUNK, ECHUNK)]

        pltpu.sync_copy(z128_ref, zbuf)
        pltpu.sync_copy(ones_ref, ones_v)
        for i in range(nsl):
            pltpu.sync_copy(zbuf, degacc.at[pl.ds(rbase + i * ECHUNK, ECHUNK)])
        plsc.subcore_barrier()

        pltpu.async_copy(dsl(0), didx_a, sem_a)

        def body(t, carry):
            j0 = 2 * t
            pltpu.async_copy(dsl(j0 + 1), didx_b, sem_b)
            pltpu.make_async_copy(dsl(j0), didx_a, sem_a).wait()
            pltpu.sync_copy(ones_v, degacc.at[didx_a], add=True)
            pltpu.async_copy(dsl(j0 + 2), didx_a, sem_a)
            pltpu.make_async_copy(dsl(j0 + 1), didx_b, sem_b).wait()
            pltpu.sync_copy(ones_v, degacc.at[didx_b], add=True)
            return carry

        lax.fori_loop(0, (NCH - 1) // 2, body, 0)
        pltpu.make_async_copy(dsl(NCH - 1), didx_a, sem_a).wait()
        pltpu.sync_copy(ones_v, degacc.at[didx_a], add=True)
        plsc.subcore_barrier()

        for i in range(nsl):
            r0 = rbase + i * ECHUNK
            pltpu.sync_copy(degacc.at[pl.ds(r0, ECHUNK)], zbuf)
            pltpu.sync_copy(zbuf, deg_out.at[c, pl.ds(r0, ECHUNK)])

    return sc_kernel(dst, z128, ones128)


def _full(shape):
    return pl.BlockSpec(shape, lambda i: tuple(0 for _ in shape))


def _dot(a, b):
    return jnp.dot(a, b, preferred_element_type=F32)


def _sage_dense(h_parts, agg_parts, deg_src, Wl, bl, Wr, relu,
                pre_projected=False, proj_W=None):
    npart = len(h_parts)
    nagg = len(agg_parts)
    aw = [a.shape[2] for a in agg_parts]
    d_out = Wl.shape[1] if not pre_projected else 128
    nout = d_out // 128

    def body(*refs):
        it = iter(refs)
        h_refs = [next(it) for _ in range(npart)]
        agg_refs = [next(it) for _ in range(nagg)]
        deg_ref = next(it)
        wl_ref = next(it) if not pre_projected else None
        bl_ref = next(it)
        wr_ref = next(it)
        pw_ref = next(it) if proj_W is not None else None
        o_refs = [next(it) for _ in range(nout + (1 if proj_W is not None else 0))]

        degs = deg_ref[0, :, 0:1] + deg_ref[1, :, 0:1]
        degc = jnp.maximum(degs, 1.0)
        acc = jnp.broadcast_to(bl_ref[...], (BR, d_out))
        for t in range(nagg):
            a = agg_refs[t][...]
            mean_t = (a[0, :, :128] + a[1, :, :128]) / degc
            if pre_projected:
                acc = acc + mean_t
            else:
                acc = acc + _dot(mean_t, wl_ref[pl.ds(t * 128, 128), :])
        for t in range(npart):
            acc = acc + _dot(h_refs[t][...], wr_ref[pl.ds(t * 128, 128), :])
        if relu:
            acc = jnp.maximum(acc, 0.0)
        for t in range(nout):
            o_refs[t][...] = acc[:, t * 128:(t + 1) * 128]
        if proj_W is not None:
            pw = pw_ref[...]
            o_refs[nout][...] = _dot(acc, pw)

    d_in = 128 * npart
    in_specs = (
        [pl.BlockSpec((BR, 128), lambda i: (i, 0)) for _ in range(npart)]
        + [pl.BlockSpec((2, BR, w), lambda i: (0, i, 0)) for w in aw]
        + [pl.BlockSpec((2, BR, 128), lambda i: (0, i, 0))]
        + ([_full((d_in, d_out))] if not pre_projected else [])
        + [_full((1, d_out)), _full((d_in, d_out))]
        + ([_full((d_out, 128))] if proj_W is not None else [])
    )
    nout_t = nout + (1 if proj_W is not None else 0)
    out_specs = [pl.BlockSpec((BR, 128), lambda i: (i, 0)) for _ in range(nout_t)]
    out_shape = [jax.ShapeDtypeStruct((N, 128), F32) for _ in range(nout_t)]
    args = (list(h_parts) + list(agg_parts) + [deg_src]
            + ([Wl] if not pre_projected else [])
            + [bl.reshape(1, d_out), Wr]
            + ([proj_W] if proj_W is not None else []))
    outs = pl.pallas_call(
        body, grid=(NB,), in_specs=in_specs, out_specs=out_specs,
        out_shape=out_shape,
    )(*args)
    return list(outs) if isinstance(outs, (list, tuple)) else [outs]


def _matmul128(h_parts, W):
    npart = len(h_parts)

    def body(*refs):
        it = iter(refs)
        h_refs = [next(it) for _ in range(npart)]
        w_ref = next(it)
        o_ref = next(it)
        acc = jnp.zeros((BR, 128), F32)
        for t in range(npart):
            acc = acc + _dot(h_refs[t][...], w_ref[pl.ds(t * 128, 128), :])
        o_ref[...] = acc

    return pl.pallas_call(
        body, grid=(NB,),
        in_specs=[pl.BlockSpec((BR, 128), lambda i: (i, 0))
                  for _ in range(npart)] + [_full((128 * npart, 128))],
        out_specs=pl.BlockSpec((BR, 128), lambda i: (i, 0)),
        out_shape=jax.ShapeDtypeStruct((N, 128), F32),
    )(*h_parts, W)


def _fcin(x, W1, b1, W2, b2):
    def body(x_ref, w1, b1r, w2, b2r, o_ref):
        h = jnp.maximum(_dot(x_ref[...], w1[...]) + b1r[...], 0.0)
        o_ref[...] = _dot(h, w2[...]) + b2r[...]

    return pl.pallas_call(
        body, grid=(NB,),
        in_specs=[pl.BlockSpec((BR, 128), lambda i: (i, 0)),
                  _full((128, 256)), _full((1, 256)),
                  _full((256, 256)), _full((1, 256))],
        out_specs=pl.BlockSpec((BR, 256), lambda i: (i, 0)),
        out_shape=jax.ShapeDtypeStruct((N, 256), F32),
    )(x, W1, b1.reshape(1, 256), W2, b2.reshape(1, 256))


def _cavg_kv(g, ids_f, Wk, bk, Wv, bv):
    def body(g_ref, ids_ref, wk, bkr, wv, bvr, k_out, v_out, logc_out,
             sums, crow, ccol):
        i = pl.program_id(0)

        @pl.when(i == 0)
        def _init():
            sums[...] = jnp.zeros_like(sums)
            crow[...] = jnp.zeros_like(crow)
            ccol[...] = jnp.zeros_like(ccol)

        ids = ids_ref[0]
        iota = lax.broadcasted_iota(jnp.int32, (BR, C), 1).astype(F32)
        oh = (ids == iota).astype(F32)
        gv = g_ref[...]
        sums[...] += lax.dot_general(oh, gv, (((0,), (0,)), ((), ())),
                                     preferred_element_type=F32)
        crow[0:1, :] += jnp.sum(oh, axis=0)[None, :]
        ccol[...] += lax.dot_general(oh, jnp.ones((BR, 8), F32),
                                     (((0,), (0,)), ((), ())),
                                     preferred_element_type=F32)

        @pl.when(i == NB - 1)
        def _fin():
            sizes = jnp.maximum(ccol[:, 0:1], 1.0)
            cavg = sums[...] / sizes
            k_out[...] = _dot(cavg, wk[...]) + bkr[...]
            v_out[...] = _dot(cavg, wv[...]) + bvr[...]
            logc_out[...] = jnp.broadcast_to(jnp.log(crow[0:1, :]), (8, C))

    return pl.pallas_call(
        body, grid=(NB,),
        in_specs=[pl.BlockSpec((BR, 256), lambda i: (i, 0)),
                  pl.BlockSpec((1, BR, 1), lambda i: (i, 0, 0)),
                  _full((256, 256)), _full((1, 256)),
                  _full((256, 256)), _full((1, 256))],
        out_specs=[_full((C, 256)), _full((C, 256)), _full((8, C))],
        out_shape=[jax.ShapeDtypeStruct((C, 256), F32),
                   jax.ShapeDtypeStruct((C, 256), F32),
                   jax.ShapeDtypeStruct((8, C), F32)],
        scratch_shapes=[pltpu.VMEM((C, 256), F32),
                        pltpu.VMEM((8, C), F32),
                        pltpu.VMEM((C, 8), F32)],
    )(g, ids_f, Wk, bk.reshape(1, 256), Wv, bv.reshape(1, 256))


def _attn_ff(g, dm, wb, kmat, vmat, logc, Wp, bp, Wq, bq, W1, b1, W2, b2):
    def body(g_ref, dm_ref, wb_ref, k_ref, v_ref, logc_ref,
             wp, bpr, wq, bqr, w1, b1r, w2, b2r, o_ref):
        gv = g_ref[...]
        qx = _dot(gv, wp[...]) + bpr[...]
        q = _dot(qx, wq[...]) + bqr[...]
        wbv = wb_ref[...]
        bias = dm_ref[...] * wbv[0:1, 0:1] + wbv[0:1, 1:2] + logc_ref[0:1, :]
        kk = k_ref[...]
        vv = v_ref[...]
        outs = []
        for h in range(HEADS):
            lo, hi = h * DHEAD, (h + 1) * DHEAD
            qh = q[:, lo:hi]
            kh = kk[:, lo:hi]
            vh = vv[:, lo:hi]
            dots = lax.dot_general(qh, kh, (((1,), (1,)), ((), ())),
                                   preferred_element_type=F32) * SCALE + bias
            m = jnp.max(dots, axis=1, keepdims=True)
            e = jnp.exp(dots - m)
            ssum = jnp.sum(e, axis=1, keepdims=True)
            outs.append(_dot(e / ssum, vh))
        o = jnp.concatenate(outs, axis=1)
        hff = jnp.maximum(_dot(o, w1[...]) + b1r[...], 0.0)
        o_ref[...] = jnp.maximum(_dot(hff, w2[...]) + b2r[...], 0.0)

    return pl.pallas_call(
        body, grid=(NB,),
        in_specs=[pl.BlockSpec((BR, 256), lambda i: (i, 0)),
                  pl.BlockSpec((BR, C), lambda i: (i, 0)),
                  _full((1, 2)),
                  _full((C, 256)), _full((C, 256)), _full((8, C)),
                  _full((256, 256)), _full((1, 256)),
                  _full((256, 256)), _full((1, 256)),
                  _full((256, 256)), _full((1, 256)),
                  _full((256, 256)), _full((1, 256))],
        out_specs=pl.BlockSpec((BR, 256), lambda i: (i, 0)),
        out_shape=jax.ShapeDtypeStruct((N, 256), F32),
    )(g, dm, wb, kmat, vmat, logc,
      Wp, bp.reshape(1, 256), Wq, bq.reshape(1, 256),
      W1, b1.reshape(1, 256), W2, b2.reshape(1, 256))


def _fcout_add(g, W, b, x_local):
    def body(g_ref, w, br, xl_ref, o_ref):
        o_ref[...] = _dot(g_ref[...], w[...]) + br[...] + xl_ref[...]

    return pl.pallas_call(
        body, grid=(NB,),
        in_specs=[pl.BlockSpec((BR, 256), lambda i: (i, 0)),
                  _full((256, 128)), _full((1, 128)),
                  pl.BlockSpec((BR, 128), lambda i: (i, 0))],
        out_specs=pl.BlockSpec((BR, 128), lambda i: (i, 0)),
        out_shape=jax.ShapeDtypeStruct((N, 128), F32),
    )(g, W, b.reshape(1, 128), x_local)


def kernel(x, edge_index, distance_matrix, nodes_to_community, params):
    src = edge_index[0]
    dst = edge_index[1]

    p = params['fc_in']
    g = _fcin(x, p['W1'], p['b1'], p['W2'], p['b2'])
    ids_f = nodes_to_community.astype(F32).reshape(NB, BR, 1)
    for li in range(len(params['convs'])):
        cp = params['convs'][li]
        fp = params['ffs'][li]
        kmat, vmat, logc = _cavg_kv(g, ids_f, cp['Wk'], cp['bk'],
                                    cp['Wv'], cp['bv'])
        wb = jnp.stack([cp['w_dis'], cp['b_dis']]).reshape(1, 2)
        g = _attn_ff(g, distance_matrix, wb, kmat, vmat, logc,
                     cp['Wp'], cp['bp'], cp['Wq'], cp['bq'],
                     fp['W1'], fp['b1'], fp['W2'], fp['b2'])

    gnn = params['gnn']
    deg = _sc_deg(dst)
    agg0 = _sc_edge_agg([x], src, dst)
    h_parts = _sage_dense([x], agg0, deg, gnn[0]['Wl'], gnn[0]['bl'],
                          gnn[0]['Wr'], relu=True)
    agg1 = _sc_edge_agg(h_parts, src, dst)
    outs1 = _sage_dense(h_parts, agg1, deg, gnn[1]['Wl'], gnn[1]['bl'],
                        gnn[1]['Wr'], relu=True, proj_W=gnn[2]['Wl'])
    h_parts = outs1[:2]
    p2 = outs1[2]
    agg2 = _sc_edge_agg([p2], src, dst)
    x_local = _sage_dense(h_parts, agg2, deg, None, gnn[2]['bl'],
                          gnn[2]['Wr'], relu=False, pre_projected=True)[0]

    op = params['fc_out']
    return _fcout_add(g, op['W'], op['b'], x_local)


# speedup vs baseline: 8.7879x; 1.0233x over previous
import functools

import jax
import jax.numpy as jnp
import numpy as np
from jax import lax
from jax.experimental import pallas as pl
from jax.experimental.pallas import tpu as pltpu
from jax.experimental.pallas import tpu_sc as plsc

N = 10000
E = 320000
C = 512
HEADS = 4
DHEAD = 64
SCALE = 1.0 / np.sqrt(DHEAD)

NB = 10
BR = N // NB

NCORES = 2
NSUB = 16
NW = NCORES * NSUB
EPT = E // NW
ECHUNK = 80
NCH = EPT // ECHUNK
NP = 10240
NRING = 2
RPT = NP // NSUB

F32 = jnp.float32


def _sc_edge_agg(h_parts, src, dst, width=128):
    nparts = len(h_parts)
    zrow = jnp.zeros((ECHUNK, width), F32)

    mesh = plsc.VectorSubcoreMesh(core_axis_name="c", subcore_axis_name="s")
    out_type = tuple(jax.ShapeDtypeStruct((NCORES, NP, width), F32)
                     for _ in range(nparts))
    scratch = (
        pltpu.VMEM((EPT,), jnp.int32),
    ) + tuple(pltpu.VMEM((ECHUNK,), jnp.int32) for _ in range(NRING)) \
      + tuple(pltpu.VMEM((ECHUNK, width), F32) for _ in range(NRING)) + (
        pltpu.VMEM((ECHUNK, width), F32),
        pltpu.VMEM_SHARED((NP, width), F32),
    ) + tuple(pltpu.SemaphoreType.DMA for _ in range(2 * NRING))

    @functools.partial(pl.kernel, mesh=mesh, out_type=out_type,
                       scratch_types=scratch)
    def sc_kernel(*refs):
        it = iter(refs)
        h_refs = [next(it) for _ in range(nparts)]
        src_ref = next(it)
        dst_ref = next(it)
        z_ref = next(it)
        agg_outs = [next(it) for _ in range(nparts)]
        sidx = next(it)
        didxs = [next(it) for _ in range(NRING)]
        rowss = [next(it) for _ in range(NRING)]
        zbuf = next(it)
        acc = next(it)
        sems = [next(it) for _ in range(NRING)]
        isems = [next(it) for _ in range(NRING)]

        c = lax.axis_index("c")
        s = lax.axis_index("s")
        wid = s * NCORES + c
        ebase = wid * EPT
        rbase = s * RPT
        nsl = RPT // ECHUNK

        pltpu.sync_copy(z_ref, zbuf)
        pltpu.sync_copy(src_ref.at[pl.ds(ebase, EPT)], sidx)

        def dsl(j):
            return dst_ref.at[pl.ds(ebase + j * ECHUNK, ECHUNK)]

        for kx in range(nparts):
            h = h_refs[kx]
            for i in range(nsl):
                pltpu.sync_copy(zbuf, acc.at[pl.ds(rbase + i * ECHUNK, ECHUNK)])
            plsc.subcore_barrier()

            def gidx(j):
                return sidx.at[pl.ds(j * ECHUNK, ECHUNK)]

            def issue(j, b, h=h):
                pltpu.async_copy(h.at[gidx(j)], rowss[b], sems[b])
                pltpu.async_copy(dsl(j), didxs[b], isems[b])

            def drain_scatter(j, b, h=h):
                pltpu.make_async_copy(h.at[gidx(j)], rowss[b], sems[b]).wait()
                pltpu.make_async_copy(dsl(j), didxs[b], isems[b]).wait()
                pltpu.sync_copy(rowss[b], acc.at[didxs[b]], add=True)

            for b in range(NRING):
                issue(b, b)

            def body(t, carry):
                j0 = NRING * t
                for b in range(NRING):
                    drain_scatter(j0 + b, b)
                    issue(j0 + NRING + b, b)
                return carry

            covered = NRING * (NCH // NRING)
            lax.fori_loop(0, NCH // NRING - 1, body, 0)
            for b in range(NRING):
                drain_scatter(covered - NRING + b, b)
            for j in range(covered, NCH):
                issue(j, 0)
                drain_scatter(j, 0)
            plsc.subcore_barrier()

            for i in range(nsl):
                r0 = rbase + i * ECHUNK
                pltpu.sync_copy(acc.at[pl.ds(r0, ECHUNK)], rowss[0])
                pltpu.sync_copy(rowss[0], agg_outs[kx].at[c, pl.ds(r0, ECHUNK)])
            if kx + 1 < nparts:
                plsc.subcore_barrier()

    outs = sc_kernel(*(list(h_parts) + [src, dst, zrow]))
    if not isinstance(outs, (list, tuple)):
        outs = (outs,)
    return list(outs)


def _sc_deg(dst):
    z128 = jnp.zeros((ECHUNK, 128), F32)
    ones128 = jnp.ones((ECHUNK, 128), F32)

    mesh = plsc.VectorSubcoreMesh(core_axis_name="c", subcore_axis_name="s")

    @functools.partial(
        pl.kernel, mesh=mesh,
        out_type=jax.ShapeDtypeStruct((NCORES, NP, 128), F32),
        scratch_types=(
            pltpu.VMEM((ECHUNK,), jnp.int32),
            pltpu.VMEM((ECHUNK,), jnp.int32),
            pltpu.VMEM((ECHUNK, 128), F32),
            pltpu.VMEM((ECHUNK, 128), F32),
            pltpu.VMEM_SHARED((NP, 128), F32),
            pltpu.SemaphoreType.DMA,
            pltpu.SemaphoreType.DMA,
        ))
    def sc_kernel(dst_ref, z128_ref, ones_ref, deg_out, didx_a, didx_b, ones_v,
                  zbuf, degacc, sem_a, sem_b):
        c = lax.axis_index("c")
        s = lax.axis_index("s")
        wid = s * NCORES + c
        ebase = wid * EPT
        rbase = s * RPT
        nsl = RPT // ECHUNK

        def dsl(j):
            return dst_ref.at[pl.ds(ebase + j * ECHUNK, ECHUNK)]

        pltpu.sync_copy(z128_ref, zbuf)
        pltpu.sync_copy(ones_ref, ones_v)
        for i in range(nsl):
            pltpu.sync_copy(zbuf, degacc.at[pl.ds(rbase + i * ECHUNK, ECHUNK)])
        plsc.subcore_barrier()

        pltpu.async_copy(dsl(0), didx_a, sem_a)

        def body(t, carry):
            j0 = 2 * t
            pltpu.async_copy(dsl(j0 + 1), didx_b, sem_b)
            pltpu.make_async_copy(dsl(j0), didx_a, sem_a).wait()
            pltpu.sync_copy(ones_v, degacc.at[didx_a], add=True)
            pltpu.async_copy(dsl(j0 + 2), didx_a, sem_a)
            pltpu.make_async_copy(dsl(j0 + 1), didx_b, sem_b).wait()
            pltpu.sync_copy(ones_v, degacc.at[didx_b], add=True)
            return carry

        lax.fori_loop(0, (NCH - 1) // 2, body, 0)
        pltpu.make_async_copy(dsl(NCH - 1), didx_a, sem_a).wait()
        pltpu.sync_copy(ones_v, degacc.at[didx_a], add=True)
        plsc.subcore_barrier()

        for i in range(nsl):
            r0 = rbase + i * ECHUNK
            pltpu.sync_copy(degacc.at[pl.ds(r0, ECHUNK)], zbuf)
            pltpu.sync_copy(zbuf, deg_out.at[c, pl.ds(r0, ECHUNK)])

    return sc_kernel(dst, z128, ones128)


def _full(shape):
    return pl.BlockSpec(shape, lambda i: tuple(0 for _ in shape))


def _dot(a, b):
    return jnp.dot(a, b, preferred_element_type=F32)


def _sage_dense(h_parts, agg_parts, deg_src, Wl, bl, Wr, relu,
                pre_projected=False, proj_W=None):
    npart = len(h_parts)
    nagg = len(agg_parts)
    aw = [a.shape[2] for a in agg_parts]
    d_out = Wl.shape[1] if not pre_projected else 128
    nout = d_out // 128

    def body(*refs):
        it = iter(refs)
        h_refs = [next(it) for _ in range(npart)]
        agg_refs = [next(it) for _ in range(nagg)]
        deg_ref = next(it)
        wl_ref = next(it) if not pre_projected else None
        bl_ref = next(it)
        wr_ref = next(it)
        pw_ref = next(it) if proj_W is not None else None
        o_refs = [next(it) for _ in range(nout + (1 if proj_W is not None else 0))]

        degs = deg_ref[0, :, 0:1] + deg_ref[1, :, 0:1]
        degc = jnp.maximum(degs, 1.0)
        acc = jnp.broadcast_to(bl_ref[...], (BR, d_out))
        for t in range(nagg):
            a = agg_refs[t][...]
            mean_t = (a[0, :, :128] + a[1, :, :128]) / degc
            if pre_projected:
                acc = acc + mean_t
            else:
                acc = acc + _dot(mean_t, wl_ref[pl.ds(t * 128, 128), :])
        for t in range(npart):
            acc = acc + _dot(h_refs[t][...], wr_ref[pl.ds(t * 128, 128), :])
        if relu:
            acc = jnp.maximum(acc, 0.0)
        for t in range(nout):
            o_refs[t][...] = acc[:, t * 128:(t + 1) * 128]
        if proj_W is not None:
            pw = pw_ref[...]
            o_refs[nout][...] = _dot(acc, pw)

    d_in = 128 * npart
    in_specs = (
        [pl.BlockSpec((BR, 128), lambda i: (i, 0)) for _ in range(npart)]
        + [pl.BlockSpec((2, BR, w), lambda i: (0, i, 0)) for w in aw]
        + [pl.BlockSpec((2, BR, 128), lambda i: (0, i, 0))]
        + ([_full((d_in, d_out))] if not pre_projected else [])
        + [_full((1, d_out)), _full((d_in, d_out))]
        + ([_full((d_out, 128))] if proj_W is not None else [])
    )
    nout_t = nout + (1 if proj_W is not None else 0)
    out_specs = [pl.BlockSpec((BR, 128), lambda i: (i, 0)) for _ in range(nout_t)]
    out_shape = [jax.ShapeDtypeStruct((N, 128), F32) for _ in range(nout_t)]
    args = (list(h_parts) + list(agg_parts) + [deg_src]
            + ([Wl] if not pre_projected else [])
            + [bl.reshape(1, d_out), Wr]
            + ([proj_W] if proj_W is not None else []))
    outs = pl.pallas_call(
        body, grid=(NB,), in_specs=in_specs, out_specs=out_specs,
        out_shape=out_shape,
    )(*args)
    return list(outs) if isinstance(outs, (list, tuple)) else [outs]


def _matmul128(h_parts, W):
    npart = len(h_parts)

    def body(*refs):
        it = iter(refs)
        h_refs = [next(it) for _ in range(npart)]
        w_ref = next(it)
        o_ref = next(it)
        acc = jnp.zeros((BR, 128), F32)
        for t in range(npart):
            acc = acc + _dot(h_refs[t][...], w_ref[pl.ds(t * 128, 128), :])
        o_ref[...] = acc

    return pl.pallas_call(
        body, grid=(NB,),
        in_specs=[pl.BlockSpec((BR, 128), lambda i: (i, 0))
                  for _ in range(npart)] + [_full((128 * npart, 128))],
        out_specs=pl.BlockSpec((BR, 128), lambda i: (i, 0)),
        out_shape=jax.ShapeDtypeStruct((N, 128), F32),
    )(*h_parts, W)


def _cavg_tail(i, oh, gv, k_out, v_out, logc_out, sums, crow, ccol,
               wk, bkr, wv, bvr):
    @pl.when(i == 0)
    def _init():
        sums[...] = jnp.zeros_like(sums)
        crow[...] = jnp.zeros_like(crow)
        ccol[...] = jnp.zeros_like(ccol)

    sums[...] += lax.dot_general(oh, gv, (((0,), (0,)), ((), ())),
                                 preferred_element_type=F32)
    crow[0:1, :] += jnp.sum(oh, axis=0)[None, :]
    ccol[...] += lax.dot_general(oh, jnp.ones((BR, 8), F32),
                                 (((0,), (0,)), ((), ())),
                                 preferred_element_type=F32)

    @pl.when(i == NB - 1)
    def _fin():
        sizes = jnp.maximum(ccol[:, 0:1], 1.0)
        cavg = sums[...] / sizes
        k_out[...] = _dot(cavg, wk[...]) + bkr[...]
        v_out[...] = _dot(cavg, wv[...]) + bvr[...]
        logc_out[...] = jnp.broadcast_to(jnp.log(crow[0:1, :]), (8, C))


def _onehot(ids):
    iota = lax.broadcasted_iota(jnp.int32, (BR, C), 1).astype(F32)
    return (ids == iota).astype(F32)


def _fcin_cavg(x, ids_f, W1, b1, W2, b2, Wk, bk, Wv, bv):
    def body(x_ref, ids_ref, w1, b1r, w2, b2r, wk, bkr, wv, bvr,
             o_ref, k_out, v_out, logc_out, sums, crow, ccol):
        i = pl.program_id(0)
        h = jnp.maximum(_dot(x_ref[...], w1[...]) + b1r[...], 0.0)
        gv = _dot(h, w2[...]) + b2r[...]
        o_ref[...] = gv
        _cavg_tail(i, _onehot(ids_ref[0]), gv, k_out, v_out, logc_out,
                   sums, crow, ccol, wk, bkr, wv, bvr)

    return pl.pallas_call(
        body, grid=(NB,),
        in_specs=[pl.BlockSpec((BR, 128), lambda i: (i, 0)),
                  pl.BlockSpec((1, BR, 1), lambda i: (i, 0, 0)),
                  _full((128, 256)), _full((1, 256)),
                  _full((256, 256)), _full((1, 256)),
                  _full((256, 256)), _full((1, 256)),
                  _full((256, 256)), _full((1, 256))],
        out_specs=[pl.BlockSpec((BR, 256), lambda i: (i, 0)),
                   _full((C, 256)), _full((C, 256)), _full((8, C))],
        out_shape=[jax.ShapeDtypeStruct((N, 256), F32),
                   jax.ShapeDtypeStruct((C, 256), F32),
                   jax.ShapeDtypeStruct((C, 256), F32),
                   jax.ShapeDtypeStruct((8, C), F32)],
        scratch_shapes=[pltpu.VMEM((C, 256), F32),
                        pltpu.VMEM((8, C), F32),
                        pltpu.VMEM((C, 8), F32)],
    )(x, ids_f, W1, b1.reshape(1, 256), W2, b2.reshape(1, 256),
      Wk, bk.reshape(1, 256), Wv, bv.reshape(1, 256))


def _attn_ff(g, dm, wb, kmat, vmat, logc, Wp, bp, Wq, bq, W1, b1, W2, b2,
             ids_f=None, next_kv=None):
    fuse = next_kv is not None

    def body(*refs):
        it = iter(refs)
        g_ref = next(it)
        dm_ref = next(it)
        wb_ref = next(it)
        k_ref = next(it)
        v_ref = next(it)
        logc_ref = next(it)
        wp, bpr, wq, bqr, w1, b1r, w2, b2r = (next(it) for _ in range(8))
        if fuse:
            ids_ref = next(it)
            wk, bkr, wv, bvr = (next(it) for _ in range(4))
        o_ref = next(it)
        if fuse:
            k_out = next(it)
            v_out = next(it)
            logc_out = next(it)
            sums = next(it)
            crow = next(it)
            ccol = next(it)

        gv = g_ref[...]
        qx = _dot(gv, wp[...]) + bpr[...]
        q = _dot(qx, wq[...]) + bqr[...]
        wbv = wb_ref[...]
        bias = dm_ref[...] * wbv[0:1, 0:1] + wbv[0:1, 1:2] + logc_ref[0:1, :]
        kk = k_ref[...]
        vv = v_ref[...]
        outs = []
        for h in range(HEADS):
            lo, hi = h * DHEAD, (h + 1) * DHEAD
            qh = q[:, lo:hi]
            kh = kk[:, lo:hi]
            vh = vv[:, lo:hi]
            dots = lax.dot_general(qh, kh, (((1,), (1,)), ((), ())),
                                   preferred_element_type=F32) * SCALE + bias
            m = jnp.max(dots, axis=1, keepdims=True)
            e = jnp.exp(dots - m)
            ssum = jnp.sum(e, axis=1, keepdims=True)
            outs.append(_dot(e / ssum, vh))
        o = jnp.concatenate(outs, axis=1)
        hff = jnp.maximum(_dot(o, w1[...]) + b1r[...], 0.0)
        g_out = jnp.maximum(_dot(hff, w2[...]) + b2r[...], 0.0)
        o_ref[...] = g_out
        if fuse:
            _cavg_tail(pl.program_id(0), _onehot(ids_ref[0]), g_out,
                       k_out, v_out, logc_out, sums, crow, ccol,
                       wk, bkr, wv, bvr)

    in_specs = [pl.BlockSpec((BR, 256), lambda i: (i, 0)),
                pl.BlockSpec((BR, C), lambda i: (i, 0)),
                _full((1, 2)),
                _full((C, 256)), _full((C, 256)), _full((8, C)),
                _full((256, 256)), _full((1, 256)),
                _full((256, 256)), _full((1, 256)),
                _full((256, 256)), _full((1, 256)),
                _full((256, 256)), _full((1, 256))]
    args = [g, dm, wb, kmat, vmat, logc,
            Wp, bp.reshape(1, 256), Wq, bq.reshape(1, 256),
            W1, b1.reshape(1, 256), W2, b2.reshape(1, 256)]
    out_specs = [pl.BlockSpec((BR, 256), lambda i: (i, 0))]
    out_shape = [jax.ShapeDtypeStruct((N, 256), F32)]
    scratch = []
    if fuse:
        Wk, bk, Wv, bv = next_kv
        in_specs += [pl.BlockSpec((1, BR, 1), lambda i: (i, 0, 0)),
                     _full((256, 256)), _full((1, 256)),
                     _full((256, 256)), _full((1, 256))]
        args += [ids_f, Wk, bk.reshape(1, 256), Wv, bv.reshape(1, 256)]
        out_specs += [_full((C, 256)), _full((C, 256)), _full((8, C))]
        out_shape += [jax.ShapeDtypeStruct((C, 256), F32),
                      jax.ShapeDtypeStruct((C, 256), F32),
                      jax.ShapeDtypeStruct((8, C), F32)]
        scratch = [pltpu.VMEM((C, 256), F32),
                   pltpu.VMEM((8, C), F32),
                   pltpu.VMEM((C, 8), F32)]
    outs = pl.pallas_call(
        body, grid=(NB,), in_specs=in_specs, out_specs=out_specs,
        out_shape=out_shape, scratch_shapes=scratch,
    )(*args)
    return outs if fuse else outs[0]


def _fcout_add(g, W, b, x_local):
    def body(g_ref, w, br, xl_ref, o_ref):
        o_ref[...] = _dot(g_ref[...], w[...]) + br[...] + xl_ref[...]

    return pl.pallas_call(
        body, grid=(NB,),
        in_specs=[pl.BlockSpec((BR, 256), lambda i: (i, 0)),
                  _full((256, 128)), _full((1, 128)),
                  pl.BlockSpec((BR, 128), lambda i: (i, 0))],
        out_specs=pl.BlockSpec((BR, 128), lambda i: (i, 0)),
        out_shape=jax.ShapeDtypeStruct((N, 128), F32),
    )(g, W, b.reshape(1, 128), x_local)


def kernel(x, edge_index, distance_matrix, nodes_to_community, params):
    src = edge_index[0]
    dst = edge_index[1]

    p = params['fc_in']
    ids_f = nodes_to_community.astype(F32).reshape(NB, BR, 1)
    cp0, cp1 = params['convs']
    fp0, fp1 = params['ffs']
    g, kmat, vmat, logc = _fcin_cavg(x, ids_f, p['W1'], p['b1'], p['W2'],
                                     p['b2'], cp0['Wk'], cp0['bk'],
                                     cp0['Wv'], cp0['bv'])
    wb0 = jnp.stack([cp0['w_dis'], cp0['b_dis']]).reshape(1, 2)
    g, kmat, vmat, logc = _attn_ff(
        g, distance_matrix, wb0, kmat, vmat, logc,
        cp0['Wp'], cp0['bp'], cp0['Wq'], cp0['bq'],
        fp0['W1'], fp0['b1'], fp0['W2'], fp0['b2'],
        ids_f=ids_f, next_kv=(cp1['Wk'], cp1['bk'], cp1['Wv'], cp1['bv']))
    wb1 = jnp.stack([cp1['w_dis'], cp1['b_dis']]).reshape(1, 2)
    g = _attn_ff(g, distance_matrix, wb1, kmat, vmat, logc,
                 cp1['Wp'], cp1['bp'], cp1['Wq'], cp1['bq'],
                 fp1['W1'], fp1['b1'], fp1['W2'], fp1['b2'])

    gnn = params['gnn']
    deg = _sc_deg(dst)
    agg0 = _sc_edge_agg([x], src, dst)
    h_parts = _sage_dense([x], agg0, deg, gnn[0]['Wl'], gnn[0]['bl'],
                          gnn[0]['Wr'], relu=True)
    agg1 = _sc_edge_agg(h_parts, src, dst)
    outs1 = _sage_dense(h_parts, agg1, deg, gnn[1]['Wl'], gnn[1]['bl'],
                        gnn[1]['Wr'], relu=True, proj_W=gnn[2]['Wl'])
    h_parts = outs1[:2]
    p2 = outs1[2]
    agg2 = _sc_edge_agg([p2], src, dst)
    x_local = _sage_dense(h_parts, agg2, deg, None, gnn[2]['bl'],
                          gnn[2]['Wr'], relu=False, pre_projected=True)[0]

    op = params['fc_out']
    return _fcout_add(g, op['W'], op['b'], x_local)


# speedup vs baseline: 8.9264x; 1.0158x over previous
import functools

import jax
import jax.numpy as jnp
import numpy as np
from jax import lax
from jax.experimental import pallas as pl
from jax.experimental.pallas import tpu as pltpu
from jax.experimental.pallas import tpu_sc as plsc

N = 10000
E = 320000
C = 512
HEADS = 4
DHEAD = 64
SCALE = 1.0 / np.sqrt(DHEAD)

NB = 10
BR = N // NB

NCORES = 2
NSUB = 16
NW = NCORES * NSUB
EPT = E // NW
ECHUNK = 80
NCH = EPT // ECHUNK
NP = 10240
NRING = 2
RPT = NP // NSUB

F32 = jnp.float32


def _sc_edge_agg(h_parts, src, dst, width=128):
    nparts = len(h_parts)
    zrow = jnp.zeros((ECHUNK, width), F32)

    mesh = plsc.VectorSubcoreMesh(core_axis_name="c", subcore_axis_name="s")
    out_type = tuple(jax.ShapeDtypeStruct((NCORES, NP, width), F32)
                     for _ in range(nparts))
    scratch = (
        pltpu.VMEM((EPT,), jnp.int32),
    ) + tuple(pltpu.VMEM((ECHUNK,), jnp.int32) for _ in range(NRING)) \
      + tuple(pltpu.VMEM((ECHUNK, width), F32) for _ in range(NRING)) + (
        pltpu.VMEM((ECHUNK, width), F32),
        pltpu.VMEM_SHARED((NP, width), F32),
    ) + tuple(pltpu.SemaphoreType.DMA for _ in range(2 * NRING))

    @functools.partial(pl.kernel, mesh=mesh, out_type=out_type,
                       scratch_types=scratch)
    def sc_kernel(*refs):
        it = iter(refs)
        h_refs = [next(it) for _ in range(nparts)]
        src_ref = next(it)
        dst_ref = next(it)
        z_ref = next(it)
        agg_outs = [next(it) for _ in range(nparts)]
        sidx = next(it)
        didxs = [next(it) for _ in range(NRING)]
        rowss = [next(it) for _ in range(NRING)]
        zbuf = next(it)
        acc = next(it)
        sems = [next(it) for _ in range(NRING)]
        isems = [next(it) for _ in range(NRING)]

        c = lax.axis_index("c")
        s = lax.axis_index("s")
        wid = s * NCORES + c
        ebase = wid * EPT
        rbase = s * RPT
        nsl = RPT // ECHUNK

        pltpu.sync_copy(z_ref, zbuf)
        pltpu.sync_copy(src_ref.at[pl.ds(ebase, EPT)], sidx)

        def dsl(j):
            return dst_ref.at[pl.ds(ebase + j * ECHUNK, ECHUNK)]

        for kx in range(nparts):
            h = h_refs[kx]
            for i in range(nsl):
                pltpu.async_copy(
                    zbuf, acc.at[pl.ds(rbase + i * ECHUNK, ECHUNK)], sems[0])
            for i in range(nsl):
                pltpu.make_async_copy(
                    zbuf, acc.at[pl.ds(rbase + i * ECHUNK, ECHUNK)],
                    sems[0]).wait()
            plsc.subcore_barrier()

            def gidx(j):
                return sidx.at[pl.ds(j * ECHUNK, ECHUNK)]

            def issue(j, b, h=h):
                pltpu.async_copy(h.at[gidx(j)], rowss[b], sems[b])
                pltpu.async_copy(dsl(j), didxs[b], isems[b])

            def drain_scatter(j, b, h=h):
                pltpu.make_async_copy(h.at[gidx(j)], rowss[b], sems[b]).wait()
                pltpu.make_async_copy(dsl(j), didxs[b], isems[b]).wait()
                pltpu.sync_copy(rowss[b], acc.at[didxs[b]], add=True)

            for b in range(NRING):
                issue(b, b)

            def body(t, carry):
                j0 = NRING * t
                for b in range(NRING):
                    drain_scatter(j0 + b, b)
                    issue(j0 + NRING + b, b)
                return carry

            covered = NRING * (NCH // NRING)
            lax.fori_loop(0, NCH // NRING - 1, body, 0)
            for b in range(NRING):
                drain_scatter(covered - NRING + b, b)
            for j in range(covered, NCH):
                issue(j, 0)
                drain_scatter(j, 0)
            plsc.subcore_barrier()

            def oslice(i, kx=kx):
                return agg_outs[kx].at[c, pl.ds(rbase + i * ECHUNK, ECHUNK)]

            for i in range(nsl):
                b = i % 2
                if i >= 2:
                    pltpu.make_async_copy(rowss[b], oslice(i - 2),
                                          isems[b]).wait()
                pltpu.sync_copy(acc.at[pl.ds(rbase + i * ECHUNK, ECHUNK)],
                                rowss[b])
                pltpu.async_copy(rowss[b], oslice(i), isems[b])
            for i in range(nsl - 2, nsl):
                b = i % 2
                pltpu.make_async_copy(rowss[b], oslice(i), isems[b]).wait()
            if kx + 1 < nparts:
                plsc.subcore_barrier()

    outs = sc_kernel(*(list(h_parts) + [src, dst, zrow]))
    if not isinstance(outs, (list, tuple)):
        outs = (outs,)
    return list(outs)


def _sc_deg(dst):
    z128 = jnp.zeros((ECHUNK, 128), F32)
    ones128 = jnp.ones((ECHUNK, 128), F32)

    mesh = plsc.VectorSubcoreMesh(core_axis_name="c", subcore_axis_name="s")

    @functools.partial(
        pl.kernel, mesh=mesh,
        out_type=jax.ShapeDtypeStruct((NCORES, NP, 128), F32),
        scratch_types=(
            pltpu.VMEM((ECHUNK,), jnp.int32),
            pltpu.VMEM((ECHUNK,), jnp.int32),
            pltpu.VMEM((ECHUNK, 128), F32),
            pltpu.VMEM((ECHUNK, 128), F32),
            pltpu.VMEM_SHARED((NP, 128), F32),
            pltpu.SemaphoreType.DMA,
            pltpu.SemaphoreType.DMA,
        ))
    def sc_kernel(dst_ref, z128_ref, ones_ref, deg_out, didx_a, didx_b, ones_v,
                  zbuf, degacc, sem_a, sem_b):
        c = lax.axis_index("c")
        s = lax.axis_index("s")
        wid = s * NCORES + c
        ebase = wid * EPT
        rbase = s * RPT
        nsl = RPT // ECHUNK

        def dsl(j):
            return dst_ref.at[pl.ds(ebase + j * ECHUNK, ECHUNK)]

        pltpu.sync_copy(z128_ref, zbuf)
        pltpu.sync_copy(ones_ref, ones_v)
        for i in range(nsl):
            pltpu.sync_copy(zbuf, degacc.at[pl.ds(rbase + i * ECHUNK, ECHUNK)])
        plsc.subcore_barrier()

        pltpu.async_copy(dsl(0), didx_a, sem_a)

        def body(t, carry):
            j0 = 2 * t
            pltpu.async_copy(dsl(j0 + 1), didx_b, sem_b)
            pltpu.make_async_copy(dsl(j0), didx_a, sem_a).wait()
            pltpu.sync_copy(ones_v, degacc.at[didx_a], add=True)
            pltpu.async_copy(dsl(j0 + 2), didx_a, sem_a)
            pltpu.make_async_copy(dsl(j0 + 1), didx_b, sem_b).wait()
            pltpu.sync_copy(ones_v, degacc.at[didx_b], add=True)
            return carry

        lax.fori_loop(0, (NCH - 1) // 2, body, 0)
        pltpu.make_async_copy(dsl(NCH - 1), didx_a, sem_a).wait()
        pltpu.sync_copy(ones_v, degacc.at[didx_a], add=True)
        plsc.subcore_barrier()

        fbufs = [zbuf, ones_v]

        def oslice(i):
            return deg_out.at[c, pl.ds(rbase + i * ECHUNK, ECHUNK)]

        fsems = [sem_a, sem_b]
        for i in range(nsl):
            b = i % 2
            if i >= 2:
                pltpu.make_async_copy(fbufs[b], oslice(i - 2), fsems[b]).wait()
            pltpu.sync_copy(degacc.at[pl.ds(rbase + i * ECHUNK, ECHUNK)],
                            fbufs[b])
            pltpu.async_copy(fbufs[b], oslice(i), fsems[b])
        for i in range(nsl - 2, nsl):
            b = i % 2
            pltpu.make_async_copy(fbufs[b], oslice(i), fsems[b]).wait()

    return sc_kernel(dst, z128, ones128)


def _full(shape):
    return pl.BlockSpec(shape, lambda i: tuple(0 for _ in shape))


def _dot(a, b):
    return jnp.dot(a, b, preferred_element_type=F32)


def _sage_dense(h_parts, agg_parts, deg_src, Wl, bl, Wr, relu,
                pre_projected=False, proj_W=None):
    npart = len(h_parts)
    nagg = len(agg_parts)
    aw = [a.shape[2] for a in agg_parts]
    d_out = Wl.shape[1] if not pre_projected else 128
    nout = d_out // 128

    def body(*refs):
        it = iter(refs)
        h_refs = [next(it) for _ in range(npart)]
        agg_refs = [next(it) for _ in range(nagg)]
        deg_ref = next(it)
        wl_ref = next(it) if not pre_projected else None
        bl_ref = next(it)
        wr_ref = next(it)
        pw_ref = next(it) if proj_W is not None else None
        o_refs = [next(it) for _ in range(nout + (1 if proj_W is not None else 0))]

        degs = deg_ref[0, :, 0:1] + deg_ref[1, :, 0:1]
        degc = jnp.maximum(degs, 1.0)
        acc = jnp.broadcast_to(bl_ref[...], (BR, d_out))
        for t in range(nagg):
            a = agg_refs[t][...]
            mean_t = (a[0, :, :128] + a[1, :, :128]) / degc
            if pre_projected:
                acc = acc + mean_t
            else:
                acc = acc + _dot(mean_t, wl_ref[pl.ds(t * 128, 128), :])
        for t in range(npart):
            acc = acc + _dot(h_refs[t][...], wr_ref[pl.ds(t * 128, 128), :])
        if relu:
            acc = jnp.maximum(acc, 0.0)
        for t in range(nout):
            o_refs[t][...] = acc[:, t * 128:(t + 1) * 128]
        if proj_W is not None:
            pw = pw_ref[...]
            o_refs[nout][...] = _dot(acc, pw)

    d_in = 128 * npart
    in_specs = (
        [pl.BlockSpec((BR, 128), lambda i: (i, 0)) for _ in range(npart)]
        + [pl.BlockSpec((2, BR, w), lambda i: (0, i, 0)) for w in aw]
        + [pl.BlockSpec((2, BR, 128), lambda i: (0, i, 0))]
        + ([_full((d_in, d_out))] if not pre_projected else [])
        + [_full((1, d_out)), _full((d_in, d_out))]
        + ([_full((d_out, 128))] if proj_W is not None else [])
    )
    nout_t = nout + (1 if proj_W is not None else 0)
    out_specs = [pl.BlockSpec((BR, 128), lambda i: (i, 0)) for _ in range(nout_t)]
    out_shape = [jax.ShapeDtypeStruct((N, 128), F32) for _ in range(nout_t)]
    args = (list(h_parts) + list(agg_parts) + [deg_src]
            + ([Wl] if not pre_projected else [])
            + [bl.reshape(1, d_out), Wr]
            + ([proj_W] if proj_W is not None else []))
    outs = pl.pallas_call(
        body, grid=(NB,), in_specs=in_specs, out_specs=out_specs,
        out_shape=out_shape,
    )(*args)
    return list(outs) if isinstance(outs, (list, tuple)) else [outs]


def _matmul128(h_parts, W):
    npart = len(h_parts)

    def body(*refs):
        it = iter(refs)
        h_refs = [next(it) for _ in range(npart)]
        w_ref = next(it)
        o_ref = next(it)
        acc = jnp.zeros((BR, 128), F32)
        for t in range(npart):
            acc = acc + _dot(h_refs[t][...], w_ref[pl.ds(t * 128, 128), :])
        o_ref[...] = acc

    return pl.pallas_call(
        body, grid=(NB,),
        in_specs=[pl.BlockSpec((BR, 128), lambda i: (i, 0))
                  for _ in range(npart)] + [_full((128 * npart, 128))],
        out_specs=pl.BlockSpec((BR, 128), lambda i: (i, 0)),
        out_shape=jax.ShapeDtypeStruct((N, 128), F32),
    )(*h_parts, W)


def _cavg_tail(i, oh, gv, k_out, v_out, logc_out, sums, crow, ccol,
               wk, bkr, wv, bvr):
    @pl.when(i == 0)
    def _init():
        sums[...] = jnp.zeros_like(sums)
        crow[...] = jnp.zeros_like(crow)
        ccol[...] = jnp.zeros_like(ccol)

    sums[...] += lax.dot_general(oh, gv, (((0,), (0,)), ((), ())),
                                 preferred_element_type=F32)
    crow[0:1, :] += jnp.sum(oh, axis=0)[None, :]
    ccol[...] += lax.dot_general(oh, jnp.ones((BR, 8), F32),
                                 (((0,), (0,)), ((), ())),
                                 preferred_element_type=F32)

    @pl.when(i == NB - 1)
    def _fin():
        sizes = jnp.maximum(ccol[:, 0:1], 1.0)
        cavg = sums[...] / sizes
        k_out[...] = _dot(cavg, wk[...]) + bkr[...]
        v_out[...] = _dot(cavg, wv[...]) + bvr[...]
        logc_out[...] = jnp.broadcast_to(jnp.log(crow[0:1, :]), (8, C))


def _onehot(ids):
    iota = lax.broadcasted_iota(jnp.int32, (BR, C), 1).astype(F32)
    return (ids == iota).astype(F32)


def _fcin_cavg(x, ids_f, W1, b1, W2, b2, Wk, bk, Wv, bv):
    def body(x_ref, ids_ref, w1, b1r, w2, b2r, wk, bkr, wv, bvr,
             o_ref, k_out, v_out, logc_out, sums, crow, ccol):
        i = pl.program_id(0)
        h = jnp.maximum(_dot(x_ref[...], w1[...]) + b1r[...], 0.0)
        gv = _dot(h, w2[...]) + b2r[...]
        o_ref[...] = gv
        _cavg_tail(i, _onehot(ids_ref[0]), gv, k_out, v_out, logc_out,
                   sums, crow, ccol, wk, bkr, wv, bvr)

    return pl.pallas_call(
        body, grid=(NB,),
        in_specs=[pl.BlockSpec((BR, 128), lambda i: (i, 0)),
                  pl.BlockSpec((1, BR, 1), lambda i: (i, 0, 0)),
                  _full((128, 256)), _full((1, 256)),
                  _full((256, 256)), _full((1, 256)),
                  _full((256, 256)), _full((1, 256)),
                  _full((256, 256)), _full((1, 256))],
        out_specs=[pl.BlockSpec((BR, 256), lambda i: (i, 0)),
                   _full((C, 256)), _full((C, 256)), _full((8, C))],
        out_shape=[jax.ShapeDtypeStruct((N, 256), F32),
                   jax.ShapeDtypeStruct((C, 256), F32),
                   jax.ShapeDtypeStruct((C, 256), F32),
                   jax.ShapeDtypeStruct((8, C), F32)],
        scratch_shapes=[pltpu.VMEM((C, 256), F32),
                        pltpu.VMEM((8, C), F32),
                        pltpu.VMEM((C, 8), F32)],
    )(x, ids_f, W1, b1.reshape(1, 256), W2, b2.reshape(1, 256),
      Wk, bk.reshape(1, 256), Wv, bv.reshape(1, 256))


def _attn_ff(g, dm, wb, kmat, vmat, logc, Wp, bp, Wq, bq, W1, b1, W2, b2,
             ids_f=None, next_kv=None):
    fuse = next_kv is not None

    def body(*refs):
        it = iter(refs)
        g_ref = next(it)
        dm_ref = next(it)
        wb_ref = next(it)
        k_ref = next(it)
        v_ref = next(it)
        logc_ref = next(it)
        wp, bpr, wq, bqr, w1, b1r, w2, b2r = (next(it) for _ in range(8))
        if fuse:
            ids_ref = next(it)
            wk, bkr, wv, bvr = (next(it) for _ in range(4))
        o_ref = next(it)
        if fuse:
            k_out = next(it)
            v_out = next(it)
            logc_out = next(it)
            sums = next(it)
            crow = next(it)
            ccol = next(it)

        gv = g_ref[...]
        qx = _dot(gv, wp[...]) + bpr[...]
        q = _dot(qx, wq[...]) + bqr[...]
        wbv = wb_ref[...]
        bias = dm_ref[...] * wbv[0:1, 0:1] + wbv[0:1, 1:2] + logc_ref[0:1, :]
        kk = k_ref[...]
        vv = v_ref[...]
        outs = []
        for h in range(HEADS):
            lo, hi = h * DHEAD, (h + 1) * DHEAD
            qh = q[:, lo:hi]
            kh = kk[:, lo:hi]
            vh = vv[:, lo:hi]
            dots = lax.dot_general(qh, kh, (((1,), (1,)), ((), ())),
                                   preferred_element_type=F32) * SCALE + bias
            m = jnp.max(dots, axis=1, keepdims=True)
            e = jnp.exp(dots - m)
            ssum = jnp.sum(e, axis=1, keepdims=True)
            outs.append(_dot(e / ssum, vh))
        o = jnp.concatenate(outs, axis=1)
        hff = jnp.maximum(_dot(o, w1[...]) + b1r[...], 0.0)
        g_out = jnp.maximum(_dot(hff, w2[...]) + b2r[...], 0.0)
        o_ref[...] = g_out
        if fuse:
            _cavg_tail(pl.program_id(0), _onehot(ids_ref[0]), g_out,
                       k_out, v_out, logc_out, sums, crow, ccol,
                       wk, bkr, wv, bvr)

    in_specs = [pl.BlockSpec((BR, 256), lambda i: (i, 0)),
                pl.BlockSpec((BR, C), lambda i: (i, 0)),
                _full((1, 2)),
                _full((C, 256)), _full((C, 256)), _full((8, C)),
                _full((256, 256)), _full((1, 256)),
                _full((256, 256)), _full((1, 256)),
                _full((256, 256)), _full((1, 256)),
                _full((256, 256)), _full((1, 256))]
    args = [g, dm, wb, kmat, vmat, logc,
            Wp, bp.reshape(1, 256), Wq, bq.reshape(1, 256),
            W1, b1.reshape(1, 256), W2, b2.reshape(1, 256)]
    out_specs = [pl.BlockSpec((BR, 256), lambda i: (i, 0))]
    out_shape = [jax.ShapeDtypeStruct((N, 256), F32)]
    scratch = []
    if fuse:
        Wk, bk, Wv, bv = next_kv
        in_specs += [pl.BlockSpec((1, BR, 1), lambda i: (i, 0, 0)),
                     _full((256, 256)), _full((1, 256)),
                     _full((256, 256)), _full((1, 256))]
        args += [ids_f, Wk, bk.reshape(1, 256), Wv, bv.reshape(1, 256)]
        out_specs += [_full((C, 256)), _full((C, 256)), _full((8, C))]
        out_shape += [jax.ShapeDtypeStruct((C, 256), F32),
                      jax.ShapeDtypeStruct((C, 256), F32),
                      jax.ShapeDtypeStruct((8, C), F32)]
        scratch = [pltpu.VMEM((C, 256), F32),
                   pltpu.VMEM((8, C), F32),
                   pltpu.VMEM((C, 8), F32)]
    outs = pl.pallas_call(
        body, grid=(NB,), in_specs=in_specs, out_specs=out_specs,
        out_shape=out_shape, scratch_shapes=scratch,
    )(*args)
    return outs if fuse else outs[0]


def _fcout_add(g, W, b, x_local):
    def body(g_ref, w, br, xl_ref, o_ref):
        o_ref[...] = _dot(g_ref[...], w[...]) + br[...] + xl_ref[...]

    return pl.pallas_call(
        body, grid=(NB,),
        in_specs=[pl.BlockSpec((BR, 256), lambda i: (i, 0)),
                  _full((256, 128)), _full((1, 128)),
                  pl.BlockSpec((BR, 128), lambda i: (i, 0))],
        out_specs=pl.BlockSpec((BR, 128), lambda i: (i, 0)),
        out_shape=jax.ShapeDtypeStruct((N, 128), F32),
    )(g, W, b.reshape(1, 128), x_local)


def kernel(x, edge_index, distance_matrix, nodes_to_community, params):
    src = edge_index[0]
    dst = edge_index[1]

    p = params['fc_in']
    ids_f = nodes_to_community.astype(F32).reshape(NB, BR, 1)
    cp0, cp1 = params['convs']
    fp0, fp1 = params['ffs']
    g, kmat, vmat, logc = _fcin_cavg(x, ids_f, p['W1'], p['b1'], p['W2'],
                                     p['b2'], cp0['Wk'], cp0['bk'],
                                     cp0['Wv'], cp0['bv'])
    wb0 = jnp.stack([cp0['w_dis'], cp0['b_dis']]).reshape(1, 2)
    g, kmat, vmat, logc = _attn_ff(
        g, distance_matrix, wb0, kmat, vmat, logc,
        cp0['Wp'], cp0['bp'], cp0['Wq'], cp0['bq'],
        fp0['W1'], fp0['b1'], fp0['W2'], fp0['b2'],
        ids_f=ids_f, next_kv=(cp1['Wk'], cp1['bk'], cp1['Wv'], cp1['bv']))
    wb1 = jnp.stack([cp1['w_dis'], cp1['b_dis']]).reshape(1, 2)
    g = _attn_ff(g, distance_matrix, wb1, kmat, vmat, logc,
                 cp1['Wp'], cp1['bp'], cp1['Wq'], cp1['bq'],
                 fp1['W1'], fp1['b1'], fp1['W2'], fp1['b2'])

    gnn = params['gnn']
    deg = _sc_deg(dst)
    agg0 = _sc_edge_agg([x], src, dst)
    h_parts = _sage_dense([x], agg0, deg, gnn[0]['Wl'], gnn[0]['bl'],
                          gnn[0]['Wr'], relu=True)
    agg1 = _sc_edge_agg(h_parts, src, dst)
    outs1 = _sage_dense(h_parts, agg1, deg, gnn[1]['Wl'], gnn[1]['bl'],
                        gnn[1]['Wr'], relu=True, proj_W=gnn[2]['Wl'])
    h_parts = outs1[:2]
    p2 = outs1[2]
    agg2 = _sc_edge_agg([p2], src, dst)
    x_local = _sage_dense(h_parts, agg2, deg, None, gnn[2]['bl'],
                          gnn[2]['Wr'], relu=False, pre_projected=True)[0]

    op = params['fc_out']
    return _fcout_add(g, op['W'], op['b'], x_local)
